# Initial kernel scaffold; baseline (speedup 1.0000x reference)
import jax
import jax.numpy as jnp
from jax.experimental import pallas as pl


def kernel(x, circle_index, parallel_node_index, batch_idx, edge_index, W_emb, b_emb, Wk, bk, Wself, bself, Wpar, bpar, bn1_g, bn1_b, mlpW1, mlpb1, mlpW2, mlpb2, bn2_g, bn2_b, oW1, ob1, oW2, ob2, oW3, ob3):
    raise NotImplementedError("write your pallas kernel here")


# speedup vs baseline: 2.5972x; 2.5972x over previous
import functools

import jax
import jax.numpy as jnp
from jax import lax
from jax.experimental import pallas as pl
from jax.experimental.pallas import tpu as pltpu
from jax.experimental.pallas import tpu_sc as plsc

N = 10000
H = 128
C = 8
K = 3
G = 400
Bn = 1000
Nb = N // Bn
NTOT = N * C + N
WORKERS = 32
PW = 2816
NTOT_PAD = WORKERS * PW
RC = 352
NCHUNK = PW // RC

_f32 = jnp.float32


@functools.cache
def _sc_gather_fn():
    @functools.partial(
        pl.kernel,
        mesh=plsc.VectorSubcoreMesh(core_axis_name="c", subcore_axis_name="s"),
        out_type=jax.ShapeDtypeStruct((NTOT_PAD, H), _f32),
        scratch_types=[
            pltpu.VMEM((RC,), jnp.int32),
            pltpu.VMEM((RC, H), _f32),
            pltpu.SemaphoreType.DMA,
        ],
    )
    def _sc_gather(h_hbm, idx_hbm, out_hbm, idx_v, rows_v, sem):
        wid = lax.axis_index("s") * 2 + lax.axis_index("c")
        base = wid * PW
        for k in range(NCHUNK):
            off = base + k * RC
            pltpu.sync_copy(idx_hbm.at[pl.ds(off, RC)], idx_v)
            pltpu.async_copy(h_hbm.at[idx_v], rows_v, sem).wait()
            pltpu.sync_copy(rows_v, out_hbm.at[pl.ds(off, RC)])

    return _sc_gather


def _embed_body(x_ref, w_ref, b_ref, o_ref):
    o_ref[...] = jnp.dot(x_ref[...], w_ref[...],
                         preferred_element_type=_f32) + b_ref[...]


def _embed(x, W, b2d):
    return pl.pallas_call(
        _embed_body,
        grid=(Nb,),
        in_specs=[
            pl.BlockSpec((Bn, H), lambda i: (i, 0)),
            pl.BlockSpec((H, H), lambda i: (0, 0)),
            pl.BlockSpec((1, H), lambda i: (0, 0)),
        ],
        out_specs=pl.BlockSpec((Bn, H), lambda i: (i, 0)),
        out_shape=jax.ShapeDtypeStruct((N, H), _f32),
    )(x, W, b2d)


def _elu(v):
    return jnp.where(v > 0, v, jnp.exp(jnp.minimum(v, 0.0)) - 1.0)


def _layer_a_body(gn_ref, gp_ref, h_ref, wk_ref, bk_ref, ws_ref, wp_ref,
                  bsp_ref, g_ref, st_ref):
    i = pl.program_id(0)
    neigh = gn_ref[...]
    acc = jnp.zeros((Bn, C, H), _f32)
    for j in range(K):
        y = jnp.dot(neigh, wk_ref[j], preferred_element_type=_f32)
        y3 = y.reshape(Bn, C, H)
        if j:
            y3 = jnp.concatenate([y3[:, j:], y3[:, :j]], axis=1)
        acc = acc + y3
    acc = acc + bk_ref[...].reshape(1, 1, H)
    agg = _elu(acc).sum(axis=1)
    h = h_ref[...]
    pre = (agg + jnp.dot(h, ws_ref[...], preferred_element_type=_f32)
           + jnp.dot(gp_ref[...], wp_ref[...], preferred_element_type=_f32)
           + bsp_ref[...])
    g = _elu(pre) + h
    g_ref[...] = g

    @pl.when(i == 0)
    def _():
        st_ref[...] = jnp.zeros((8, H), _f32)

    st_ref[0:1, :] += g.sum(axis=0, keepdims=True)
    st_ref[1:2, :] += (g * g).sum(axis=0, keepdims=True)


def _layer_a(gath, h, Wk_l, bk_sum, Ws_l, Wp_l, bsp):
    return pl.pallas_call(
        _layer_a_body,
        grid=(Nb,),
        in_specs=[
            pl.BlockSpec((Bn * C, H), lambda i: (i, 0)),
            pl.BlockSpec((Bn, H), lambda i: (i + N * C // Bn, 0)),
            pl.BlockSpec((Bn, H), lambda i: (i, 0)),
            pl.BlockSpec((K, H, H), lambda i: (0, 0, 0)),
            pl.BlockSpec((1, H), lambda i: (0, 0)),
            pl.BlockSpec((H, H), lambda i: (0, 0)),
            pl.BlockSpec((H, H), lambda i: (0, 0)),
            pl.BlockSpec((1, H), lambda i: (0, 0)),
        ],
        out_specs=[
            pl.BlockSpec((Bn, H), lambda i: (i, 0)),
            pl.BlockSpec((8, H), lambda i: (0, 0)),
        ],
        out_shape=[
            jax.ShapeDtypeStruct((N, H), _f32),
            jax.ShapeDtypeStruct((8, H), _f32),
        ],
    )(gath, gath, h, Wk_l, bk_sum, Ws_l, Wp_l, bsp)


def _bn_scale(st_ref, gam_ref, bet_ref):
    mean = st_ref[0:1, :] * (1.0 / N)
    var = st_ref[1:2, :] * (1.0 / N) - mean * mean
    inv = lax.rsqrt(var + 1e-5) * gam_ref[...]
    return inv, bet_ref[...] - mean * inv


def _layer_b_body(g_ref, st_ref, gam_ref, bet_ref, w1_ref, b1_ref, w2_ref,
                  b2_ref, u_ref, st2_ref):
    i = pl.program_id(0)
    sc, sh = _bn_scale(st_ref, gam_ref, bet_ref)
    gn = g_ref[...] * sc + sh
    m = jnp.maximum(jnp.dot(gn, w1_ref[...], preferred_element_type=_f32)
                    + b1_ref[...], 0.0)
    m = jnp.dot(m, w2_ref[...], preferred_element_type=_f32) + b2_ref[...]
    u = gn + m
    u_ref[...] = u

    @pl.when(i == 0)
    def _():
        st2_ref[...] = jnp.zeros((8, H), _f32)

    st2_ref[0:1, :] += u.sum(axis=0, keepdims=True)
    st2_ref[1:2, :] += (u * u).sum(axis=0, keepdims=True)


def _layer_b(g, st1, gam, bet, W1, b1, W2, b2):
    return pl.pallas_call(
        _layer_b_body,
        grid=(Nb,),
        in_specs=[
            pl.BlockSpec((Bn, H), lambda i: (i, 0)),
            pl.BlockSpec((8, H), lambda i: (0, 0)),
            pl.BlockSpec((1, H), lambda i: (0, 0)),
            pl.BlockSpec((1, H), lambda i: (0, 0)),
            pl.BlockSpec((H, H), lambda i: (0, 0)),
            pl.BlockSpec((1, H), lambda i: (0, 0)),
            pl.BlockSpec((H, H), lambda i: (0, 0)),
            pl.BlockSpec((1, H), lambda i: (0, 0)),
        ],
        out_specs=[
            pl.BlockSpec((Bn, H), lambda i: (i, 0)),
            pl.BlockSpec((8, H), lambda i: (0, 0)),
        ],
        out_shape=[
            jax.ShapeDtypeStruct((N, H), _f32),
            jax.ShapeDtypeStruct((8, H), _f32),
        ],
    )(g, st1, gam, bet, W1, b1, W2, b2)


def _layer_c_body(u_ref, st_ref, gam_ref, bet_ref, h_ref):
    sc, sh = _bn_scale(st_ref, gam_ref, bet_ref)
    h_ref[...] = u_ref[...] * sc + sh


def _layer_c(u, st2, gam, bet):
    return pl.pallas_call(
        _layer_c_body,
        grid=(Nb,),
        in_specs=[
            pl.BlockSpec((Bn, H), lambda i: (i, 0)),
            pl.BlockSpec((8, H), lambda i: (0, 0)),
            pl.BlockSpec((1, H), lambda i: (0, 0)),
            pl.BlockSpec((1, H), lambda i: (0, 0)),
        ],
        out_specs=pl.BlockSpec((Bn, H), lambda i: (i, 0)),
        out_shape=jax.ShapeDtypeStruct((N, H), _f32),
    )(u, st2, gam, bet)


def _pool_body(h_ref, bi_ref, w1_ref, b1_ref, w2_ref, b2_ref, w3_ref, b3_ref,
               pooled_ref, z_ref):
    i = pl.program_id(0)

    @pl.when(i == 0)
    def _():
        pooled_ref[...] = jnp.zeros((G, H), _f32)

    bi = bi_ref[...].reshape(1, Bn)
    oh = (lax.broadcasted_iota(jnp.int32, (G, Bn), 0) == bi).astype(_f32)
    pooled_ref[...] += jnp.dot(oh, h_ref[...], preferred_element_type=_f32)

    @pl.when(i == Nb - 1)
    def _():
        p = pooled_ref[...]
        z1 = jnp.maximum(jnp.dot(p, w1_ref[...], preferred_element_type=_f32)
                         + b1_ref[...], 0.0)
        z2 = jnp.maximum(jnp.dot(z1, w2_ref[...], preferred_element_type=_f32)
                         + b2_ref[...], 0.0)
        z_ref[...] = ((z2 * w3_ref[...]).sum(axis=1, keepdims=True)
                      + b3_ref[...])


def _pool_out(h, bidx3, oW1, ob1, oW2, ob2, oW3row, ob3s):
    pooled, z = pl.pallas_call(
        _pool_body,
        grid=(Nb,),
        in_specs=[
            pl.BlockSpec((Bn, H), lambda i: (i, 0)),
            pl.BlockSpec((1, 1, Bn), lambda i: (i, 0, 0)),
            pl.BlockSpec((H, H // 2), lambda i: (0, 0)),
            pl.BlockSpec((1, H // 2), lambda i: (0, 0)),
            pl.BlockSpec((H // 2, H // 4), lambda i: (0, 0)),
            pl.BlockSpec((1, H // 4), lambda i: (0, 0)),
            pl.BlockSpec((1, H // 4), lambda i: (0, 0)),
            pl.BlockSpec((1, 1), lambda i: (0, 0)),
        ],
        out_specs=[
            pl.BlockSpec((G, H), lambda i: (0, 0)),
            pl.BlockSpec((G, 1), lambda i: (0, 0)),
        ],
        out_shape=[
            jax.ShapeDtypeStruct((G, H), _f32),
            jax.ShapeDtypeStruct((G, 1), _f32),
        ],
    )(h, bidx3, oW1, ob1, oW2, ob2, oW3row, ob3s)
    del pooled
    return z


def kernel(x, circle_index, parallel_node_index, batch_idx, edge_index,
           W_emb, b_emb, Wk, bk, Wself, bself, Wpar, bpar,
           bn1_g, bn1_b, mlpW1, mlpb1, mlpW2, mlpb2, bn2_g, bn2_b,
           oW1, ob1, oW2, ob2, oW3, ob3):
    del edge_index
    idx_all = jnp.concatenate([
        circle_index.reshape(-1).astype(jnp.int32),
        parallel_node_index.astype(jnp.int32),
        jnp.zeros((NTOT_PAD - NTOT,), jnp.int32),
    ])
    bidx3 = batch_idx.astype(jnp.int32).reshape(Nb, 1, Bn)
    bk_sum = bk.sum(axis=1)
    bsp = bself + bpar

    h = _embed(x, W_emb, b_emb.reshape(1, H))
    for l in range(Wk.shape[0]):
        gath = _sc_gather_fn()(h, idx_all)
        g, st1 = _layer_a(gath, h, Wk[l], bk_sum[l].reshape(1, H),
                          Wself[l], Wpar[l], bsp[l].reshape(1, H))
        u, st2 = _layer_b(g, st1, bn1_g[l].reshape(1, H),
                          bn1_b[l].reshape(1, H), mlpW1[l],
                          mlpb1[l].reshape(1, H), mlpW2[l],
                          mlpb2[l].reshape(1, H))
        h = _layer_c(u, st2, bn2_g[l].reshape(1, H), bn2_b[l].reshape(1, H))
    return _pool_out(h, bidx3, oW1, ob1.reshape(1, H // 2),
                     oW2, ob2.reshape(1, H // 4), oW3.reshape(1, H // 4),
                     ob3.reshape(1, 1))


# speedup vs baseline: 3.1783x; 1.2238x over previous
import functools

import jax
import jax.numpy as jnp
from jax import lax
from jax.experimental import pallas as pl
from jax.experimental.pallas import tpu as pltpu
from jax.experimental.pallas import tpu_sc as plsc

N = 10000
H = 128
C = 8
K = 3
G = 400
Bn = 1000
Nb = N // Bn
NBH = Nb // 2
GRP = Bn * C + Bn
HALF = NBH * GRP
WORKERS = 32
PW = 1408
HALF_PAD = WORKERS * PW
RC = 352
NCHUNK = PW // RC

_f32 = jnp.float32


@functools.cache
def _sc_gather_fn():
    @functools.partial(
        pl.kernel,
        mesh=plsc.VectorSubcoreMesh(core_axis_name="c", subcore_axis_name="s"),
        out_type=jax.ShapeDtypeStruct((HALF_PAD, H), _f32),
        scratch_types=[
            pltpu.VMEM((RC,), jnp.int32),
            pltpu.VMEM((RC, H), _f32),
            pltpu.SemaphoreType.DMA,
        ],
    )
    def _sc_gather(u_hbm, idx_hbm, out_hbm, idx_v, rows_v, sem):
        wid = lax.axis_index("s") * 2 + lax.axis_index("c")
        base = wid * PW
        for k in range(NCHUNK):
            off = base + k * RC
            pltpu.sync_copy(idx_hbm.at[pl.ds(off, RC)], idx_v)
            pltpu.async_copy(u_hbm.at[idx_v], rows_v, sem).wait()
            pltpu.sync_copy(rows_v, out_hbm.at[pl.ds(off, RC)])

    return _sc_gather


def _embed_body(x_ref, w_ref, b_ref, o_ref):
    o_ref[...] = jnp.dot(x_ref[...], w_ref[...],
                         preferred_element_type=_f32) + b_ref[...]


def _embed(x, W, b2d):
    return pl.pallas_call(
        _embed_body,
        grid=(Nb,),
        in_specs=[
            pl.BlockSpec((Bn, H), lambda i: (i, 0)),
            pl.BlockSpec((H, H), lambda i: (0, 0)),
            pl.BlockSpec((1, H), lambda i: (0, 0)),
        ],
        out_specs=pl.BlockSpec((Bn, H), lambda i: (i, 0)),
        out_shape=jax.ShapeDtypeStruct((N, H), _f32),
    )(x, W, b2d)


def _elu(v):
    return jnp.where(v > 0, v, jnp.exp(jnp.minimum(v, 0.0)) - 1.0)


def _bn_scale(st_ref, gam_ref, bet_ref):
    mean = st_ref[0:1, :] * (1.0 / N)
    var = st_ref[1:2, :] * (1.0 / N) - mean * mean
    sc = lax.rsqrt(var + 1e-5) * gam_ref[...]
    return sc, bet_ref[...] - mean * sc


def _layer_a_body(gath_ref, u_ref, st_ref, gam_ref, bet_ref, wk_ref, bk_ref,
                  ws_ref, wp_ref, bsp_ref, g_ref, st1_ref):
    i = pl.program_id(0)
    sc, sh = _bn_scale(st_ref, gam_ref, bet_ref)
    blk = gath_ref[...] * sc + sh
    neigh = blk[:Bn * C, :]
    hp = blk[Bn * C:, :]
    h = u_ref[...] * sc + sh
    acc = jnp.zeros((Bn, C, H), _f32)
    for j in range(K):
        y = jnp.dot(neigh, wk_ref[j], preferred_element_type=_f32)
        y3 = y.reshape(Bn, C, H)
        if j:
            y3 = jnp.concatenate([y3[:, j:], y3[:, :j]], axis=1)
        acc = acc + y3
    acc = acc + bk_ref[...].reshape(1, 1, H)
    agg = _elu(acc).sum(axis=1)
    pre = (agg + jnp.dot(h, ws_ref[...], preferred_element_type=_f32)
           + jnp.dot(hp, wp_ref[...], preferred_element_type=_f32)
           + bsp_ref[...])
    g = _elu(pre) + h
    g_ref[...] = g

    @pl.when(i == 0)
    def _():
        st1_ref[...] = jnp.zeros((8, H), _f32)

    st1_ref[0:1, :] += g.sum(axis=0, keepdims=True)
    st1_ref[1:2, :] += (g * g).sum(axis=0, keepdims=True)


def _layer_a_half(gath, u, st_prev, gam_prev, bet_prev, Wk_l, bk_sum, Ws_l,
                  Wp_l, bsp, half):
    return pl.pallas_call(
        _layer_a_body,
        grid=(NBH,),
        in_specs=[
            pl.BlockSpec((GRP, H), lambda i: (i, 0)),
            pl.BlockSpec((Bn, H), lambda i: (i + half * NBH, 0)),
            pl.BlockSpec((8, H), lambda i: (0, 0)),
            pl.BlockSpec((1, H), lambda i: (0, 0)),
            pl.BlockSpec((1, H), lambda i: (0, 0)),
            pl.BlockSpec((K, H, H), lambda i: (0, 0, 0)),
            pl.BlockSpec((1, H), lambda i: (0, 0)),
            pl.BlockSpec((H, H), lambda i: (0, 0)),
            pl.BlockSpec((H, H), lambda i: (0, 0)),
            pl.BlockSpec((1, H), lambda i: (0, 0)),
        ],
        out_specs=[
            pl.BlockSpec((Bn, H), lambda i: (i, 0)),
            pl.BlockSpec((8, H), lambda i: (0, 0)),
        ],
        out_shape=[
            jax.ShapeDtypeStruct((NBH * Bn, H), _f32),
            jax.ShapeDtypeStruct((8, H), _f32),
        ],
    )(gath, u, st_prev, gam_prev, bet_prev, Wk_l, bk_sum, Ws_l, Wp_l, bsp)


def _layer_b_body(ga_ref, gb_ref, sta_ref, stb_ref, gam_ref, bet_ref, w1_ref,
                  b1_ref, w2_ref, b2_ref, u_ref, st2_ref):
    i = pl.program_id(0)
    st = sta_ref[...] + stb_ref[...]
    mean = st[0:1, :] * (1.0 / N)
    var = st[1:2, :] * (1.0 / N) - mean * mean
    sc = lax.rsqrt(var + 1e-5) * gam_ref[...]
    sh = bet_ref[...] - mean * sc
    g = jnp.where(i < NBH, ga_ref[...], gb_ref[...])
    gn = g * sc + sh
    m = jnp.maximum(jnp.dot(gn, w1_ref[...], preferred_element_type=_f32)
                    + b1_ref[...], 0.0)
    m = jnp.dot(m, w2_ref[...], preferred_element_type=_f32) + b2_ref[...]
    u = gn + m
    u_ref[...] = u

    @pl.when(i == 0)
    def _():
        st2_ref[...] = jnp.zeros((8, H), _f32)

    st2_ref[0:1, :] += u.sum(axis=0, keepdims=True)
    st2_ref[1:2, :] += (u * u).sum(axis=0, keepdims=True)


def _layer_b(gA, gB, stA, stB, gam, bet, W1, b1, W2, b2):
    return pl.pallas_call(
        _layer_b_body,
        grid=(Nb,),
        in_specs=[
            pl.BlockSpec((Bn, H), lambda i: (jnp.minimum(i, NBH - 1), 0)),
            pl.BlockSpec((Bn, H), lambda i: (jnp.maximum(i - NBH, 0), 0)),
            pl.BlockSpec((8, H), lambda i: (0, 0)),
            pl.BlockSpec((8, H), lambda i: (0, 0)),
            pl.BlockSpec((1, H), lambda i: (0, 0)),
            pl.BlockSpec((1, H), lambda i: (0, 0)),
            pl.BlockSpec((H, H), lambda i: (0, 0)),
            pl.BlockSpec((1, H), lambda i: (0, 0)),
            pl.BlockSpec((H, H), lambda i: (0, 0)),
            pl.BlockSpec((1, H), lambda i: (0, 0)),
        ],
        out_specs=[
            pl.BlockSpec((Bn, H), lambda i: (i, 0)),
            pl.BlockSpec((8, H), lambda i: (0, 0)),
        ],
        out_shape=[
            jax.ShapeDtypeStruct((N, H), _f32),
            jax.ShapeDtypeStruct((8, H), _f32),
        ],
    )(gA, gB, stA, stB, gam, bet, W1, b1, W2, b2)


def _pool_body(u_ref, st_ref, gam_ref, bet_ref, bi_ref, w1_ref, b1_ref,
               w2_ref, b2_ref, w3_ref, b3_ref, pooled_ref, z_ref):
    i = pl.program_id(0)

    @pl.when(i == 0)
    def _():
        pooled_ref[...] = jnp.zeros((G, H), _f32)

    sc, sh = _bn_scale(st_ref, gam_ref, bet_ref)
    h = u_ref[...] * sc + sh
    bi = bi_ref[...].reshape(1, Bn)
    oh = (lax.broadcasted_iota(jnp.int32, (G, Bn), 0) == bi).astype(_f32)
    pooled_ref[...] += jnp.dot(oh, h, preferred_element_type=_f32)

    @pl.when(i == Nb - 1)
    def _():
        p = pooled_ref[...]
        z1 = jnp.maximum(jnp.dot(p, w1_ref[...], preferred_element_type=_f32)
                         + b1_ref[...], 0.0)
        z2 = jnp.maximum(jnp.dot(z1, w2_ref[...], preferred_element_type=_f32)
                         + b2_ref[...], 0.0)
        z_ref[...] = ((z2 * w3_ref[...]).sum(axis=1, keepdims=True)
                      + b3_ref[...])


def _pool_out(u, st2, gam, bet, bidx3, oW1, ob1, oW2, ob2, oW3row, ob3s):
    pooled, z = pl.pallas_call(
        _pool_body,
        grid=(Nb,),
        in_specs=[
            pl.BlockSpec((Bn, H), lambda i: (i, 0)),
            pl.BlockSpec((8, H), lambda i: (0, 0)),
            pl.BlockSpec((1, H), lambda i: (0, 0)),
            pl.BlockSpec((1, H), lambda i: (0, 0)),
            pl.BlockSpec((1, 1, Bn), lambda i: (i, 0, 0)),
            pl.BlockSpec((H, H // 2), lambda i: (0, 0)),
            pl.BlockSpec((1, H // 2), lambda i: (0, 0)),
            pl.BlockSpec((H // 2, H // 4), lambda i: (0, 0)),
            pl.BlockSpec((1, H // 4), lambda i: (0, 0)),
            pl.BlockSpec((1, H // 4), lambda i: (0, 0)),
            pl.BlockSpec((1, 1), lambda i: (0, 0)),
        ],
        out_specs=[
            pl.BlockSpec((G, H), lambda i: (0, 0)),
            pl.BlockSpec((G, 1), lambda i: (0, 0)),
        ],
        out_shape=[
            jax.ShapeDtypeStruct((G, H), _f32),
            jax.ShapeDtypeStruct((G, 1), _f32),
        ],
    )(u, st2, gam, bet, bidx3, oW1, ob1, oW2, ob2, oW3row, ob3s)
    del pooled
    return z


def kernel(x, circle_index, parallel_node_index, batch_idx, edge_index,
           W_emb, b_emb, Wk, bk, Wself, bself, Wpar, bpar,
           bn1_g, bn1_b, mlpW1, mlpb1, mlpW2, mlpb2, bn2_g, bn2_b,
           oW1, ob1, oW2, ob2, oW3, ob3):
    del edge_index
    cid2 = circle_index.astype(jnp.int32).reshape(Nb, Bn * C)
    par2 = parallel_node_index.astype(jnp.int32).reshape(Nb, Bn)
    packed = jnp.concatenate([cid2, par2], axis=1)
    pad = jnp.zeros((HALF_PAD - HALF,), jnp.int32)
    idxA = jnp.concatenate([packed[:NBH].reshape(-1), pad])
    idxB = jnp.concatenate([packed[NBH:].reshape(-1), pad])
    bidx3 = batch_idx.astype(jnp.int32).reshape(Nb, 1, Bn)
    bk_sum = bk.sum(axis=1)
    bsp = bself + bpar
    ones1 = jnp.ones((1, H), _f32)
    zeros1 = jnp.zeros((1, H), _f32)
    stats_id = jnp.concatenate(
        [jnp.zeros((1, H), _f32),
         jnp.full((1, H), N * (1.0 - 1e-5), _f32),
         jnp.zeros((6, H), _f32)])

    u = _embed(x, W_emb, b_emb.reshape(1, H))
    st_prev, gam_prev, bet_prev = stats_id, ones1, zeros1
    gather = _sc_gather_fn()
    for l in range(Wk.shape[0]):
        gthA = gather(u, idxA)
        gthB = gather(u, idxB)
        gA, stA = _layer_a_half(gthA, u, st_prev, gam_prev, bet_prev, Wk[l],
                                bk_sum[l].reshape(1, H), Wself[l], Wpar[l],
                                bsp[l].reshape(1, H), 0)
        gB, stB = _layer_a_half(gthB, u, st_prev, gam_prev, bet_prev, Wk[l],
                                bk_sum[l].reshape(1, H), Wself[l], Wpar[l],
                                bsp[l].reshape(1, H), 1)
        u, st2 = _layer_b(gA, gB, stA, stB, bn1_g[l].reshape(1, H),
                          bn1_b[l].reshape(1, H), mlpW1[l],
                          mlpb1[l].reshape(1, H), mlpW2[l],
                          mlpb2[l].reshape(1, H))
        st_prev, gam_prev, bet_prev = (st2, bn2_g[l].reshape(1, H),
                                       bn2_b[l].reshape(1, H))
    return _pool_out(u, st_prev, gam_prev, bet_prev, bidx3, oW1,
                     ob1.reshape(1, H // 2), oW2, ob2.reshape(1, H // 4),
                     oW3.reshape(1, H // 4), ob3.reshape(1, 1))


# speedup vs baseline: 3.2068x; 1.0090x over previous
import functools

import jax
import jax.numpy as jnp
from jax import lax
from jax.experimental import pallas as pl
from jax.experimental.pallas import tpu as pltpu
from jax.experimental.pallas import tpu_sc as plsc

N = 10000
H = 128
C = 8
K = 3
G = 400
Bn = 1000
Nb = N // Bn
NBH = Nb // 2
GRP = Bn * C + Bn + 8
HALF = NBH * GRP
WORKERS = 32
PW = 1408
HALF_PAD = WORKERS * PW
RC = 352
NCHUNK = PW // RC

_f32 = jnp.float32
_bf16 = jnp.bfloat16


@functools.cache
def _sc_gather_fn():
    @functools.partial(
        pl.kernel,
        mesh=plsc.VectorSubcoreMesh(core_axis_name="c", subcore_axis_name="s"),
        out_type=jax.ShapeDtypeStruct((HALF_PAD, H), _f32),
        scratch_types=[
            pltpu.VMEM((PW,), jnp.int32),
            pltpu.VMEM((RC, H), _f32),
            pltpu.VMEM((RC, H), _f32),
            pltpu.SemaphoreType.DMA,
            pltpu.SemaphoreType.DMA,
            pltpu.SemaphoreType.DMA,
            pltpu.SemaphoreType.DMA,
        ],
    )
    def _sc_gather(u_hbm, idx_hbm, out_hbm, idx_v, rows0, rows1, gs0, gs1,
                   ws0, ws1):
        wid = lax.axis_index("s") * 2 + lax.axis_index("c")
        base = wid * PW
        bufs = [rows0, rows1]
        gsems = [gs0, gs1]
        wsems = [ws0, ws1]
        pltpu.sync_copy(idx_hbm.at[pl.ds(base, PW)], idx_v)
        gath = [None] * NCHUNK
        wr = [None] * NCHUNK
        gath[0] = pltpu.async_copy(
            u_hbm.at[idx_v.at[pl.ds(0, RC)]], bufs[0], gsems[0])
        for k in range(NCHUNK):
            gath[k].wait()
            if k + 1 < NCHUNK:
                if k >= 1:
                    wr[k - 1].wait()
                gath[k + 1] = pltpu.async_copy(
                    u_hbm.at[idx_v.at[pl.ds((k + 1) * RC, RC)]],
                    bufs[(k + 1) % 2], gsems[(k + 1) % 2])
            wr[k] = pltpu.async_copy(
                bufs[k % 2], out_hbm.at[pl.ds(base + k * RC, RC)],
                wsems[k % 2])
        wr[NCHUNK - 2].wait()
        wr[NCHUNK - 1].wait()

    return _sc_gather


def _embed_body(x_ref, w_ref, b_ref, o_ref):
    o_ref[...] = jnp.dot(x_ref[...], w_ref[...],
                         preferred_element_type=_f32) + b_ref[...]


def _embed(x, W, b2d):
    return pl.pallas_call(
        _embed_body,
        grid=(Nb,),
        in_specs=[
            pl.BlockSpec((Bn, H), lambda i: (i, 0)),
            pl.BlockSpec((H, H), lambda i: (0, 0)),
            pl.BlockSpec((1, H), lambda i: (0, 0)),
        ],
        out_specs=pl.BlockSpec((Bn, H), lambda i: (i, 0)),
        out_shape=jax.ShapeDtypeStruct((N, H), _f32),
    )(x, W, b2d)


def _elu(v):
    return jnp.where(v > 0, v, jnp.exp(jnp.minimum(v, 0.0)) - 1.0)


def _bn_scale(st_ref, gam_ref, bet_ref):
    mean = st_ref[0:1, :] * (1.0 / N)
    var = st_ref[1:2, :] * (1.0 / N) - mean * mean
    sc = lax.rsqrt(var + 1e-5) * gam_ref[...]
    return sc, bet_ref[...] - mean * sc


def _layer_a_body(gath_ref, u_ref, st_ref, gam_ref, bet_ref, wk_ref, bk_ref,
                  ws_ref, wp_ref, bsp_ref, g_ref, st1_ref):
    i = pl.program_id(0)
    sc, sh = _bn_scale(st_ref, gam_ref, bet_ref)
    blk = gath_ref[...] * sc + sh
    neigh = blk[:Bn * C, :]
    hp = blk[Bn * C:Bn * C + Bn, :]
    h = u_ref[...] * sc + sh
    acc = jnp.zeros((Bn, C, H), _f32)
    for j in range(K):
        y = jnp.dot(neigh, wk_ref[j], preferred_element_type=_f32)
        y3 = y.reshape(Bn, C, H)
        if j:
            y3 = jnp.concatenate([y3[:, j:], y3[:, :j]], axis=1)
        acc = acc + y3
    acc = acc + bk_ref[...].reshape(1, 1, H)
    agg = _elu(acc).sum(axis=1)
    pre = (agg + jnp.dot(h, ws_ref[...], preferred_element_type=_f32)
           + jnp.dot(hp, wp_ref[...], preferred_element_type=_f32)
           + bsp_ref[...])
    g = _elu(pre) + h
    g_ref[...] = g

    @pl.when(i == 0)
    def _():
        st1_ref[...] = jnp.zeros((8, H), _f32)

    st1_ref[0:1, :] += g.sum(axis=0, keepdims=True)
    st1_ref[1:2, :] += (g * g).sum(axis=0, keepdims=True)


def _layer_a_half(gath, u, st_prev, gam_prev, bet_prev, Wk_l, bk_sum, Ws_l,
                  Wp_l, bsp, half):
    return pl.pallas_call(
        _layer_a_body,
        grid=(NBH,),
        in_specs=[
            pl.BlockSpec((GRP, H), lambda i: (i, 0)),
            pl.BlockSpec((Bn, H), lambda i: (i + half * NBH, 0)),
            pl.BlockSpec((8, H), lambda i: (0, 0)),
            pl.BlockSpec((1, H), lambda i: (0, 0)),
            pl.BlockSpec((1, H), lambda i: (0, 0)),
            pl.BlockSpec((K, H, H), lambda i: (0, 0, 0)),
            pl.BlockSpec((1, H), lambda i: (0, 0)),
            pl.BlockSpec((H, H), lambda i: (0, 0)),
            pl.BlockSpec((H, H), lambda i: (0, 0)),
            pl.BlockSpec((1, H), lambda i: (0, 0)),
        ],
        out_specs=[
            pl.BlockSpec((Bn, H), lambda i: (i, 0)),
            pl.BlockSpec((8, H), lambda i: (0, 0)),
        ],
        out_shape=[
            jax.ShapeDtypeStruct((NBH * Bn, H), _f32),
            jax.ShapeDtypeStruct((8, H), _f32),
        ],
    )(gath, u, st_prev, gam_prev, bet_prev, Wk_l, bk_sum, Ws_l, Wp_l, bsp)


def _layer_b_body(ga_ref, gb_ref, sta_ref, stb_ref, gam_ref, bet_ref, w1_ref,
                  b1_ref, w2_ref, b2_ref, u_ref, st2_ref):
    i = pl.program_id(0)
    st = sta_ref[...] + stb_ref[...]
    mean = st[0:1, :] * (1.0 / N)
    var = st[1:2, :] * (1.0 / N) - mean * mean
    sc = lax.rsqrt(var + 1e-5) * gam_ref[...]
    sh = bet_ref[...] - mean * sc
    g = jnp.where(i < NBH, ga_ref[...], gb_ref[...])
    gn = g * sc + sh
    m = jnp.maximum(jnp.dot(gn, w1_ref[...], preferred_element_type=_f32)
                    + b1_ref[...], 0.0)
    m = jnp.dot(m, w2_ref[...], preferred_element_type=_f32) + b2_ref[...]
    u = gn + m
    u_ref[...] = u

    @pl.when(i == 0)
    def _():
        st2_ref[...] = jnp.zeros((8, H), _f32)

    st2_ref[0:1, :] += u.sum(axis=0, keepdims=True)
    st2_ref[1:2, :] += (u * u).sum(axis=0, keepdims=True)


def _layer_b(gA, gB, stA, stB, gam, bet, W1, b1, W2, b2):
    return pl.pallas_call(
        _layer_b_body,
        grid=(Nb,),
        in_specs=[
            pl.BlockSpec((Bn, H), lambda i: (jnp.minimum(i, NBH - 1), 0)),
            pl.BlockSpec((Bn, H), lambda i: (jnp.maximum(i - NBH, 0), 0)),
            pl.BlockSpec((8, H), lambda i: (0, 0)),
            pl.BlockSpec((8, H), lambda i: (0, 0)),
            pl.BlockSpec((1, H), lambda i: (0, 0)),
            pl.BlockSpec((1, H), lambda i: (0, 0)),
            pl.BlockSpec((H, H), lambda i: (0, 0)),
            pl.BlockSpec((1, H), lambda i: (0, 0)),
            pl.BlockSpec((H, H), lambda i: (0, 0)),
            pl.BlockSpec((1, H), lambda i: (0, 0)),
        ],
        out_specs=[
            pl.BlockSpec((Bn, H), lambda i: (i, 0)),
            pl.BlockSpec((8, H), lambda i: (0, 0)),
        ],
        out_shape=[
            jax.ShapeDtypeStruct((N, H), _f32),
            jax.ShapeDtypeStruct((8, H), _f32),
        ],
    )(gA, gB, stA, stB, gam, bet, W1, b1, W2, b2)


def _pool_body(u_ref, st_ref, gam_ref, bet_ref, bi_ref, w1_ref, b1_ref,
               w2_ref, b2_ref, w3_ref, b3_ref, pooled_ref, z_ref):
    i = pl.program_id(0)

    @pl.when(i == 0)
    def _():
        pooled_ref[...] = jnp.zeros((G, H), _f32)

    sc, sh = _bn_scale(st_ref, gam_ref, bet_ref)
    h = u_ref[...] * sc + sh
    bi = bi_ref[...].reshape(1, Bn)
    oh = (lax.broadcasted_iota(jnp.int32, (G, Bn), 0) == bi).astype(_f32)
    pooled_ref[...] += jnp.dot(oh, h, preferred_element_type=_f32)

    @pl.when(i == Nb - 1)
    def _():
        p = pooled_ref[...]
        z1 = jnp.maximum(jnp.dot(p, w1_ref[...], preferred_element_type=_f32)
                         + b1_ref[...], 0.0)
        z2 = jnp.maximum(jnp.dot(z1, w2_ref[...], preferred_element_type=_f32)
                         + b2_ref[...], 0.0)
        z_ref[...] = ((z2 * w3_ref[...]).sum(axis=1, keepdims=True)
                      + b3_ref[...])


def _pool_out(u, st2, gam, bet, bidx3, oW1, ob1, oW2, ob2, oW3row, ob3s):
    pooled, z = pl.pallas_call(
        _pool_body,
        grid=(Nb,),
        in_specs=[
            pl.BlockSpec((Bn, H), lambda i: (i, 0)),
            pl.BlockSpec((8, H), lambda i: (0, 0)),
            pl.BlockSpec((1, H), lambda i: (0, 0)),
            pl.BlockSpec((1, H), lambda i: (0, 0)),
            pl.BlockSpec((1, 1, Bn), lambda i: (i, 0, 0)),
            pl.BlockSpec((H, H // 2), lambda i: (0, 0)),
            pl.BlockSpec((1, H // 2), lambda i: (0, 0)),
            pl.BlockSpec((H // 2, H // 4), lambda i: (0, 0)),
            pl.BlockSpec((1, H // 4), lambda i: (0, 0)),
            pl.BlockSpec((1, H // 4), lambda i: (0, 0)),
            pl.BlockSpec((1, 1), lambda i: (0, 0)),
        ],
        out_specs=[
            pl.BlockSpec((G, H), lambda i: (0, 0)),
            pl.BlockSpec((G, 1), lambda i: (0, 0)),
        ],
        out_shape=[
            jax.ShapeDtypeStruct((G, H), _f32),
            jax.ShapeDtypeStruct((G, 1), _f32),
        ],
    )(u, st2, gam, bet, bidx3, oW1, ob1, oW2, ob2, oW3row, ob3s)
    del pooled
    return z


def kernel(x, circle_index, parallel_node_index, batch_idx, edge_index,
           W_emb, b_emb, Wk, bk, Wself, bself, Wpar, bpar,
           bn1_g, bn1_b, mlpW1, mlpb1, mlpW2, mlpb2, bn2_g, bn2_b,
           oW1, ob1, oW2, ob2, oW3, ob3):
    del edge_index
    cid2 = circle_index.astype(jnp.int32).reshape(Nb, Bn * C)
    par2 = parallel_node_index.astype(jnp.int32).reshape(Nb, Bn)
    packed = jnp.concatenate(
        [cid2, par2, jnp.zeros((Nb, 8), jnp.int32)], axis=1)
    pad = jnp.zeros((HALF_PAD - HALF,), jnp.int32)
    idxA = jnp.concatenate([packed[:NBH].reshape(-1), pad])
    idxB = jnp.concatenate([packed[NBH:].reshape(-1), pad])
    bidx3 = batch_idx.astype(jnp.int32).reshape(Nb, 1, Bn)
    bk_sum = bk.sum(axis=1)
    bsp = bself + bpar
    ones1 = jnp.ones((1, H), _f32)
    zeros1 = jnp.zeros((1, H), _f32)
    stats_id = jnp.concatenate(
        [jnp.zeros((1, H), _f32),
         jnp.full((1, H), N * (1.0 - 1e-5), _f32),
         jnp.zeros((6, H), _f32)])

    u = _embed(x, W_emb, b_emb.reshape(1, H))
    st_prev, gam_prev, bet_prev = stats_id, ones1, zeros1
    gather = _sc_gather_fn()
    for l in range(Wk.shape[0]):
        gthA = gather(u, idxA)
        gthB = gather(u, idxB)
        gA, stA = _layer_a_half(gthA, u, st_prev, gam_prev, bet_prev, Wk[l],
                                bk_sum[l].reshape(1, H), Wself[l], Wpar[l],
                                bsp[l].reshape(1, H), 0)
        gB, stB = _layer_a_half(gthB, u, st_prev, gam_prev, bet_prev, Wk[l],
                                bk_sum[l].reshape(1, H), Wself[l], Wpar[l],
                                bsp[l].reshape(1, H), 1)
        u, st2 = _layer_b(gA, gB, stA, stB, bn1_g[l].reshape(1, H),
                          bn1_b[l].reshape(1, H), mlpW1[l],
                          mlpb1[l].reshape(1, H), mlpW2[l],
                          mlpb2[l].reshape(1, H))
        st_prev, gam_prev, bet_prev = (st2, bn2_g[l].reshape(1, H),
                                       bn2_b[l].reshape(1, H))
    return _pool_out(u, st_prev, gam_prev, bet_prev, bidx3, oW1,
                     ob1.reshape(1, H // 2), oW2, ob2.reshape(1, H // 4),
                     oW3.reshape(1, H // 4), ob3.reshape(1, 1))


# speedup vs baseline: 3.2800x; 1.0228x over previous
import functools

import jax
import jax.numpy as jnp
from jax import lax
from jax.experimental import pallas as pl
from jax.experimental.pallas import tpu as pltpu
from jax.experimental.pallas import tpu_sc as plsc

N = 10000
H = 128
C = 8
K = 3
G = 400
Bn = 1000
Nb = N // Bn
NBH = Nb // 2
GRP = Bn * C + Bn + 8
HALF = NBH * GRP
WORKERS = 32
PW = 1408
HALF_PAD = WORKERS * PW
RC = 352
NCHUNK = PW // RC

_f32 = jnp.float32
_bf16 = jnp.bfloat16


@functools.cache
def _sc_gather_fn():
    @functools.partial(
        pl.kernel,
        mesh=plsc.VectorSubcoreMesh(core_axis_name="c", subcore_axis_name="s"),
        out_type=jax.ShapeDtypeStruct((HALF_PAD, H), _f32),
    scratch_types=[
            pltpu.VMEM((PW,), jnp.int32),
            pltpu.VMEM((RC, H), _f32),
            pltpu.VMEM((RC, H), _f32),
            pltpu.SemaphoreType.DMA,
            pltpu.SemaphoreType.DMA,
            pltpu.SemaphoreType.DMA,
            pltpu.SemaphoreType.DMA,
        ],
    )
    def _sc_gather(u_hbm, idx_hbm, out_hbm, idx_v, rows0, rows1,
                   gs0, gs1, ws0, ws1):
        wid = lax.axis_index("s") * 2 + lax.axis_index("c")
        base = wid * PW
        bufs = [rows0, rows1]
        gsems = [gs0, gs1]
        wsems = [ws0, ws1]
        pltpu.sync_copy(idx_hbm.at[pl.ds(base, PW)], idx_v)
        gath = [None] * NCHUNK
        wr = [None] * NCHUNK
        gath[0] = pltpu.async_copy(
            u_hbm.at[idx_v.at[pl.ds(0, RC)]], bufs[0], gsems[0])
        for k in range(NCHUNK):
            gath[k].wait()
            if k + 1 < NCHUNK:
                if k >= 1:
                    wr[k - 1].wait()
                gath[k + 1] = pltpu.async_copy(
                    u_hbm.at[idx_v.at[pl.ds((k + 1) * RC, RC)]],
                    bufs[(k + 1) % 2], gsems[(k + 1) % 2])
            wr[k] = pltpu.async_copy(
                bufs[k % 2], out_hbm.at[pl.ds(base + k * RC, RC)],
                wsems[k % 2])
        wr[NCHUNK - 2].wait()
        wr[NCHUNK - 1].wait()

    return _sc_gather


def _embed_body(x_ref, w_ref, b_ref, o_ref):
    o_ref[...] = jnp.dot(x_ref[...], w_ref[...],
                         preferred_element_type=_f32) + b_ref[...]


def _embed(x, W, b2d):
    return pl.pallas_call(
        _embed_body,
        grid=(Nb,),
        in_specs=[
            pl.BlockSpec((Bn, H), lambda i: (i, 0)),
            pl.BlockSpec((H, H), lambda i: (0, 0)),
            pl.BlockSpec((1, H), lambda i: (0, 0)),
        ],
        out_specs=pl.BlockSpec((Bn, H), lambda i: (i, 0)),
        out_shape=jax.ShapeDtypeStruct((N, H), _f32),
    )(x, W, b2d)


def _elu(v):
    return jnp.where(v > 0, v, jnp.exp(jnp.minimum(v, 0.0)) - 1.0)


def _bn_scale(st_ref, gam_ref, bet_ref):
    mean = st_ref[0:1, :] * (1.0 / N)
    var = st_ref[1:2, :] * (1.0 / N) - mean * mean
    sc = lax.rsqrt(var + 1e-5) * gam_ref[...]
    return sc, bet_ref[...] - mean * sc


def _layer_a_body(gath_ref, u_ref, st_ref, gam_ref, bet_ref, wk_ref, bk_ref,
                  ws_ref, wp_ref, bsp_ref, g_ref, st1_ref):
    i = pl.program_id(0)
    sc, sh = _bn_scale(st_ref, gam_ref, bet_ref)
    blk = gath_ref[...] * sc + sh
    neigh = blk[:Bn * C, :]
    hp = blk[Bn * C:Bn * C + Bn, :]
    h = u_ref[...] * sc + sh
    acc = jnp.zeros((Bn, C, H), _f32)
    for j in range(K):
        y = jnp.dot(neigh, wk_ref[j], preferred_element_type=_f32)
        y3 = y.reshape(Bn, C, H)
        if j:
            y3 = jnp.concatenate([y3[:, j:], y3[:, :j]], axis=1)
        acc = acc + y3
    acc = acc + bk_ref[...].reshape(1, 1, H)
    agg = _elu(acc).sum(axis=1)
    pre = (agg + jnp.dot(h, ws_ref[...], preferred_element_type=_f32)
           + jnp.dot(hp, wp_ref[...], preferred_element_type=_f32)
           + bsp_ref[...])
    g = _elu(pre) + h
    g_ref[...] = g

    @pl.when(i == 0)
    def _():
        st1_ref[...] = jnp.zeros((8, H), _f32)

    st1_ref[0:1, :] += g.sum(axis=0, keepdims=True)
    st1_ref[1:2, :] += (g * g).sum(axis=0, keepdims=True)


def _layer_a_half(gath, u, st_prev, gam_prev, bet_prev, Wk_l, bk_sum, Ws_l,
                  Wp_l, bsp, half):
    return pl.pallas_call(
        _layer_a_body,
        grid=(NBH,),
        in_specs=[
            pl.BlockSpec((GRP, H), lambda i: (i, 0)),
            pl.BlockSpec((Bn, H), lambda i: (i + half * NBH, 0)),
            pl.BlockSpec((8, H), lambda i: (0, 0)),
            pl.BlockSpec((1, H), lambda i: (0, 0)),
            pl.BlockSpec((1, H), lambda i: (0, 0)),
            pl.BlockSpec((K, H, H), lambda i: (0, 0, 0)),
            pl.BlockSpec((1, H), lambda i: (0, 0)),
            pl.BlockSpec((H, H), lambda i: (0, 0)),
            pl.BlockSpec((H, H), lambda i: (0, 0)),
            pl.BlockSpec((1, H), lambda i: (0, 0)),
        ],
        out_specs=[
            pl.BlockSpec((Bn, H), lambda i: (i, 0)),
            pl.BlockSpec((8, H), lambda i: (0, 0)),
        ],
        out_shape=[
            jax.ShapeDtypeStruct((NBH * Bn, H), _f32),
            jax.ShapeDtypeStruct((8, H), _f32),
        ],
    )(gath, u, st_prev, gam_prev, bet_prev, Wk_l, bk_sum, Ws_l, Wp_l, bsp)


def _bn1_mlp(hh, ii, ga_ref, gb_ref, sta_ref, stb_ref, gam_ref, bet_ref,
             w1_ref, b1_ref, w2_ref, b2_ref):
    st = sta_ref[...] + stb_ref[...]
    mean = st[0:1, :] * (1.0 / N)
    var = st[1:2, :] * (1.0 / N) - mean * mean
    sc = lax.rsqrt(var + 1e-5) * gam_ref[...]
    sh = bet_ref[...] - mean * sc
    g = jnp.where(hh == 0, ga_ref[...], gb_ref[...])
    gn = g * sc + sh
    m = jnp.maximum(jnp.dot(gn, w1_ref[...], preferred_element_type=_f32)
                    + b1_ref[...], 0.0)
    m = jnp.dot(m, w2_ref[...], preferred_element_type=_f32) + b2_ref[...]
    return gn + m


_B_IN_SPECS = [
    pl.BlockSpec((Bn, H), lambda h, i: (i * (1 - h), 0)),
    pl.BlockSpec((Bn, H), lambda h, i: (i * h, 0)),
    pl.BlockSpec((8, H), lambda h, i: (0, 0)),
    pl.BlockSpec((8, H), lambda h, i: (0, 0)),
    pl.BlockSpec((1, H), lambda h, i: (0, 0)),
    pl.BlockSpec((1, H), lambda h, i: (0, 0)),
    pl.BlockSpec((H, H), lambda h, i: (0, 0)),
    pl.BlockSpec((1, H), lambda h, i: (0, 0)),
    pl.BlockSpec((H, H), lambda h, i: (0, 0)),
    pl.BlockSpec((1, H), lambda h, i: (0, 0)),
]


def _layer_b_body(ga_ref, gb_ref, sta_ref, stb_ref, gam_ref, bet_ref, w1_ref,
                  b1_ref, w2_ref, b2_ref, u_ref, st2_ref):
    hh = pl.program_id(0)
    ii = pl.program_id(1)
    u = _bn1_mlp(hh, ii, ga_ref, gb_ref, sta_ref, stb_ref, gam_ref, bet_ref,
                 w1_ref, b1_ref, w2_ref, b2_ref)
    u_ref[...] = u

    @pl.when((hh == 0) & (ii == 0))
    def _():
        st2_ref[...] = jnp.zeros((8, H), _f32)

    st2_ref[0:1, :] += u.sum(axis=0, keepdims=True)
    st2_ref[1:2, :] += (u * u).sum(axis=0, keepdims=True)


def _layer_b(gA, gB, stA, stB, gam, bet, W1, b1, W2, b2):
    return pl.pallas_call(
        _layer_b_body,
        grid=(2, NBH),
        in_specs=_B_IN_SPECS,
        out_specs=[
            pl.BlockSpec((Bn, H), lambda h, i: (h * NBH + i, 0)),
            pl.BlockSpec((8, H), lambda h, i: (0, 0)),
        ],
        out_shape=[
            jax.ShapeDtypeStruct((N, H), _f32),
            jax.ShapeDtypeStruct((8, H), _f32),
        ],
    )(gA, gB, stA, stB, gam, bet, W1, b1, W2, b2)


def _layer_b_last_body(ga_ref, gb_ref, sta_ref, stb_ref, gam_ref, bet_ref,
                       w1_ref, b1_ref, w2_ref, b2_ref, gam2_ref, bet2_ref,
                       bi_ref, ow1_ref, ob1_ref, ow2_ref, ob2_ref, ow3_ref,
                       ob3_ref, z_ref, st2_s, pooled_s, cnt_s):
    hh = pl.program_id(0)
    ii = pl.program_id(1)
    u = _bn1_mlp(hh, ii, ga_ref, gb_ref, sta_ref, stb_ref, gam_ref, bet_ref,
                 w1_ref, b1_ref, w2_ref, b2_ref)

    @pl.when((hh == 0) & (ii == 0))
    def _():
        st2_s[...] = jnp.zeros((8, H), _f32)
        pooled_s[...] = jnp.zeros((G, H), _f32)
        cnt_s[...] = jnp.zeros((G, H), _f32)

    st2_s[0:1, :] += u.sum(axis=0, keepdims=True)
    st2_s[1:2, :] += (u * u).sum(axis=0, keepdims=True)
    bi = bi_ref[...].reshape(1, Bn)
    oh = (lax.broadcasted_iota(jnp.int32, (G, Bn), 0) == bi).astype(_f32)
    pooled_s[...] += jnp.dot(oh, u, preferred_element_type=_f32)
    cnt_s[...] += jnp.sum(oh, axis=1, keepdims=True)

    @pl.when((hh == 1) & (ii == NBH - 1))
    def _():
        st = st2_s[...]
        mean = st[0:1, :] * (1.0 / N)
        var = st[1:2, :] * (1.0 / N) - mean * mean
        sc = lax.rsqrt(var + 1e-5) * gam2_ref[...]
        sh = bet2_ref[...] - mean * sc
        p = pooled_s[...] * sc + cnt_s[...][:, 0:1] * sh
        z1 = jnp.maximum(jnp.dot(p, ow1_ref[...], preferred_element_type=_f32)
                         + ob1_ref[...], 0.0)
        z2 = jnp.maximum(jnp.dot(z1, ow2_ref[...],
                                 preferred_element_type=_f32)
                         + ob2_ref[...], 0.0)
        z_ref[...] = ((z2 * ow3_ref[...]).sum(axis=1, keepdims=True)
                      + ob3_ref[...])


def _layer_b_last(gA, gB, stA, stB, gam, bet, W1, b1, W2, b2, gam2, bet2,
                  bidx3, oW1, ob1, oW2, ob2, oW3row, ob3s):
    return pl.pallas_call(
        _layer_b_last_body,
        grid=(2, NBH),
        in_specs=_B_IN_SPECS + [
            pl.BlockSpec((1, H), lambda h, i: (0, 0)),
            pl.BlockSpec((1, H), lambda h, i: (0, 0)),
            pl.BlockSpec((1, 1, Bn), lambda h, i: (h * NBH + i, 0, 0)),
            pl.BlockSpec((H, H // 2), lambda h, i: (0, 0)),
            pl.BlockSpec((1, H // 2), lambda h, i: (0, 0)),
            pl.BlockSpec((H // 2, H // 4), lambda h, i: (0, 0)),
            pl.BlockSpec((1, H // 4), lambda h, i: (0, 0)),
            pl.BlockSpec((1, H // 4), lambda h, i: (0, 0)),
            pl.BlockSpec((1, 1), lambda h, i: (0, 0)),
        ],
        out_specs=pl.BlockSpec((G, 1), lambda h, i: (0, 0)),
        out_shape=jax.ShapeDtypeStruct((G, 1), _f32),
        scratch_shapes=[
            pltpu.VMEM((8, H), _f32),
            pltpu.VMEM((G, H), _f32),
            pltpu.VMEM((G, H), _f32),
        ],
    )(gA, gB, stA, stB, gam, bet, W1, b1, W2, b2, gam2, bet2, bidx3,
      oW1, ob1, oW2, ob2, oW3row, ob3s)


def kernel(x, circle_index, parallel_node_index, batch_idx, edge_index,
           W_emb, b_emb, Wk, bk, Wself, bself, Wpar, bpar,
           bn1_g, bn1_b, mlpW1, mlpb1, mlpW2, mlpb2, bn2_g, bn2_b,
           oW1, ob1, oW2, ob2, oW3, ob3):
    del edge_index
    cid2 = circle_index.astype(jnp.int32).reshape(Nb, Bn * C)
    par2 = parallel_node_index.astype(jnp.int32).reshape(Nb, Bn)
    packed = jnp.concatenate(
        [cid2, par2, jnp.zeros((Nb, 8), jnp.int32)], axis=1)
    pad = jnp.zeros((HALF_PAD - HALF,), jnp.int32)
    idxA = jnp.concatenate([packed[:NBH].reshape(-1), pad])
    idxB = jnp.concatenate([packed[NBH:].reshape(-1), pad])
    bidx3 = batch_idx.astype(jnp.int32).reshape(Nb, 1, Bn)
    bk_sum = bk.sum(axis=1)
    bsp = bself + bpar
    ones1 = jnp.ones((1, H), _f32)
    zeros1 = jnp.zeros((1, H), _f32)
    stats_id = jnp.concatenate(
        [jnp.zeros((1, H), _f32),
         jnp.full((1, H), N * (1.0 - 1e-5), _f32),
         jnp.zeros((6, H), _f32)])

    u = _embed(x, W_emb, b_emb.reshape(1, H))
    st_prev, gam_prev, bet_prev = stats_id, ones1, zeros1
    gather = _sc_gather_fn()
    L = Wk.shape[0]
    for l in range(L):
        gthA = gather(u, idxA)
        gthB = gather(u, idxB)
        gA, stA = _layer_a_half(gthA, u, st_prev, gam_prev, bet_prev, Wk[l],
                                bk_sum[l].reshape(1, H), Wself[l], Wpar[l],
                                bsp[l].reshape(1, H), 0)
        gB, stB = _layer_a_half(gthB, u, st_prev, gam_prev, bet_prev, Wk[l],
                                bk_sum[l].reshape(1, H), Wself[l], Wpar[l],
                                bsp[l].reshape(1, H), 1)
        bargs = (gA, gB, stA, stB, bn1_g[l].reshape(1, H),
                 bn1_b[l].reshape(1, H), mlpW1[l], mlpb1[l].reshape(1, H),
                 mlpW2[l], mlpb2[l].reshape(1, H))
        if l == L - 1:
            return _layer_b_last(*bargs, bn2_g[l].reshape(1, H),
                                 bn2_b[l].reshape(1, H), bidx3, oW1,
                                 ob1.reshape(1, H // 2), oW2,
                                 ob2.reshape(1, H // 4),
                                 oW3.reshape(1, H // 4), ob3.reshape(1, 1))
        u, st2 = _layer_b(*bargs)
        st_prev, gam_prev, bet_prev = (st2, bn2_g[l].reshape(1, H),
                                       bn2_b[l].reshape(1, H))


# speedup vs baseline: 3.3534x; 1.0224x over previous
import functools

import jax
import jax.numpy as jnp
from jax import lax
from jax.experimental import pallas as pl
from jax.experimental.pallas import tpu as pltpu
from jax.experimental.pallas import tpu_sc as plsc

N = 10000
H = 128
C = 8
K = 3
G = 400
Bn = 1000
Nb = N // Bn
NBH = Nb // 2
GRP = Bn * C + Bn + 8
HALF = NBH * GRP
WORKERS = 32
PW = 1408
HALF_PAD = WORKERS * PW
RC = 352
NCHUNK = PW // RC

_f32 = jnp.float32
_bf16 = jnp.bfloat16


@functools.cache
def _sc_gather_fn():
    @functools.partial(
        pl.kernel,
        mesh=plsc.VectorSubcoreMesh(core_axis_name="c", subcore_axis_name="s"),
        out_type=jax.ShapeDtypeStruct((HALF_PAD, H), _f32),
    scratch_types=[
            pltpu.VMEM((PW,), jnp.int32),
            pltpu.VMEM((RC, H), _f32),
            pltpu.VMEM((RC, H), _f32),
            pltpu.SemaphoreType.DMA,
            pltpu.SemaphoreType.DMA,
            pltpu.SemaphoreType.DMA,
            pltpu.SemaphoreType.DMA,
        ],
    )
    def _sc_gather(u_hbm, idx_hbm, out_hbm, idx_v, rows0, rows1,
                   gs0, gs1, ws0, ws1):
        wid = lax.axis_index("s") * 2 + lax.axis_index("c")
        base = wid * PW
        bufs = [rows0, rows1]
        gsems = [gs0, gs1]
        wsems = [ws0, ws1]
        pltpu.sync_copy(idx_hbm.at[pl.ds(base, PW)], idx_v)
        gath = [None] * NCHUNK
        wr = [None] * NCHUNK
        gath[0] = pltpu.async_copy(
            u_hbm.at[idx_v.at[pl.ds(0, RC)]], bufs[0], gsems[0])
        for k in range(NCHUNK):
            gath[k].wait()
            if k + 1 < NCHUNK:
                if k >= 1:
                    wr[k - 1].wait()
                gath[k + 1] = pltpu.async_copy(
                    u_hbm.at[idx_v.at[pl.ds((k + 1) * RC, RC)]],
                    bufs[(k + 1) % 2], gsems[(k + 1) % 2])
            wr[k] = pltpu.async_copy(
                bufs[k % 2], out_hbm.at[pl.ds(base + k * RC, RC)],
                wsems[k % 2])
        wr[NCHUNK - 2].wait()
        wr[NCHUNK - 1].wait()

    return _sc_gather


def _embed_body(x_ref, w_ref, b_ref, o_ref):
    o_ref[...] = jnp.dot(x_ref[...], w_ref[...],
                         preferred_element_type=_f32) + b_ref[...]


def _embed(x, W, b2d):
    return pl.pallas_call(
        _embed_body,
        grid=(Nb,),
        in_specs=[
            pl.BlockSpec((Bn, H), lambda i: (i, 0)),
            pl.BlockSpec((H, H), lambda i: (0, 0)),
            pl.BlockSpec((1, H), lambda i: (0, 0)),
        ],
        out_specs=pl.BlockSpec((Bn, H), lambda i: (i, 0)),
        out_shape=jax.ShapeDtypeStruct((N, H), _f32),
    )(x, W, b2d)


def _elu(v):
    return jnp.where(v > 0, v, jnp.exp(v) - 1.0)


def _bn_scale(st_ref, gam_ref, bet_ref):
    mean = st_ref[0:1, :] * (1.0 / N)
    var = st_ref[1:2, :] * (1.0 / N) - mean * mean
    sc = lax.rsqrt(var + 1e-5) * gam_ref[...]
    return sc, bet_ref[...] - mean * sc


def _layer_a_body(gath_ref, u_ref, st_ref, gam_ref, bet_ref, wk_ref, bk_ref,
                  ws_ref, wp_ref, bsp_ref, g_ref, st1_ref):
    i = pl.program_id(0)
    sc, sh = _bn_scale(st_ref, gam_ref, bet_ref)
    blk = gath_ref[...] * sc + sh
    neigh = blk[:Bn * C, :]
    hp = blk[Bn * C:Bn * C + Bn, :]
    h = u_ref[...] * sc + sh
    acc = jnp.zeros((Bn, C, H), _f32)
    for j in range(K):
        y = jnp.dot(neigh, wk_ref[j], preferred_element_type=_f32)
        y3 = y.reshape(Bn, C, H)
        if j:
            y3 = jnp.concatenate([y3[:, j:], y3[:, :j]], axis=1)
        acc = acc + y3
    acc = acc + bk_ref[...].reshape(1, 1, H)
    agg = _elu(acc).sum(axis=1)
    pre = (agg + jnp.dot(h, ws_ref[...], preferred_element_type=_f32)
           + jnp.dot(hp, wp_ref[...], preferred_element_type=_f32)
           + bsp_ref[...])
    g = _elu(pre) + h
    g_ref[...] = g

    @pl.when(i == 0)
    def _():
        st1_ref[...] = jnp.zeros((8, H), _f32)

    ones_row = jnp.full((1, Bn), 1.0, _f32)
    st1_ref[0:1, :] += jnp.dot(ones_row, g, preferred_element_type=_f32)
    st1_ref[1:2, :] += jnp.dot(ones_row, g * g, preferred_element_type=_f32)


def _layer_a_half(gath, u, st_prev, gam_prev, bet_prev, Wk_l, bk_sum, Ws_l,
                  Wp_l, bsp, half):
    return pl.pallas_call(
        _layer_a_body,
        grid=(NBH,),
        in_specs=[
            pl.BlockSpec((GRP, H), lambda i: (i, 0)),
            pl.BlockSpec((Bn, H), lambda i: (i + half * NBH, 0)),
            pl.BlockSpec((8, H), lambda i: (0, 0)),
            pl.BlockSpec((1, H), lambda i: (0, 0)),
            pl.BlockSpec((1, H), lambda i: (0, 0)),
            pl.BlockSpec((K, H, H), lambda i: (0, 0, 0)),
            pl.BlockSpec((1, H), lambda i: (0, 0)),
            pl.BlockSpec((H, H), lambda i: (0, 0)),
            pl.BlockSpec((H, H), lambda i: (0, 0)),
            pl.BlockSpec((1, H), lambda i: (0, 0)),
        ],
        out_specs=[
            pl.BlockSpec((Bn, H), lambda i: (i, 0)),
            pl.BlockSpec((8, H), lambda i: (0, 0)),
        ],
        out_shape=[
            jax.ShapeDtypeStruct((NBH * Bn, H), _f32),
            jax.ShapeDtypeStruct((8, H), _f32),
        ],
    )(gath, u, st_prev, gam_prev, bet_prev, Wk_l, bk_sum, Ws_l, Wp_l, bsp)


def _bn1_mlp(hh, ii, ga_ref, gb_ref, sta_ref, stb_ref, gam_ref, bet_ref,
             w1_ref, b1_ref, w2_ref, b2_ref):
    st = sta_ref[...] + stb_ref[...]
    mean = st[0:1, :] * (1.0 / N)
    var = st[1:2, :] * (1.0 / N) - mean * mean
    sc = lax.rsqrt(var + 1e-5) * gam_ref[...]
    sh = bet_ref[...] - mean * sc
    g = jnp.where(hh == 0, ga_ref[...], gb_ref[...])
    gn = g * sc + sh
    m = jnp.maximum(jnp.dot(gn, w1_ref[...], preferred_element_type=_f32)
                    + b1_ref[...], 0.0)
    m = jnp.dot(m, w2_ref[...], preferred_element_type=_f32) + b2_ref[...]
    return gn + m


_B_IN_SPECS = [
    pl.BlockSpec((Bn, H), lambda h, i: (i * (1 - h), 0)),
    pl.BlockSpec((Bn, H), lambda h, i: (i * h, 0)),
    pl.BlockSpec((8, H), lambda h, i: (0, 0)),
    pl.BlockSpec((8, H), lambda h, i: (0, 0)),
    pl.BlockSpec((1, H), lambda h, i: (0, 0)),
    pl.BlockSpec((1, H), lambda h, i: (0, 0)),
    pl.BlockSpec((H, H), lambda h, i: (0, 0)),
    pl.BlockSpec((1, H), lambda h, i: (0, 0)),
    pl.BlockSpec((H, H), lambda h, i: (0, 0)),
    pl.BlockSpec((1, H), lambda h, i: (0, 0)),
]


def _layer_b_body(ga_ref, gb_ref, sta_ref, stb_ref, gam_ref, bet_ref, w1_ref,
                  b1_ref, w2_ref, b2_ref, u_ref, st2_ref):
    hh = pl.program_id(0)
    ii = pl.program_id(1)
    u = _bn1_mlp(hh, ii, ga_ref, gb_ref, sta_ref, stb_ref, gam_ref, bet_ref,
                 w1_ref, b1_ref, w2_ref, b2_ref)
    u_ref[...] = u

    @pl.when((hh == 0) & (ii == 0))
    def _():
        st2_ref[...] = jnp.zeros((8, H), _f32)

    ones_row = jnp.full((1, Bn), 1.0, _f32)
    st2_ref[0:1, :] += jnp.dot(ones_row, u, preferred_element_type=_f32)
    st2_ref[1:2, :] += jnp.dot(ones_row, u * u, preferred_element_type=_f32)


def _layer_b(gA, gB, stA, stB, gam, bet, W1, b1, W2, b2):
    return pl.pallas_call(
        _layer_b_body,
        grid=(2, NBH),
        in_specs=_B_IN_SPECS,
        out_specs=[
            pl.BlockSpec((Bn, H), lambda h, i: (h * NBH + i, 0)),
            pl.BlockSpec((8, H), lambda h, i: (0, 0)),
        ],
        out_shape=[
            jax.ShapeDtypeStruct((N, H), _f32),
            jax.ShapeDtypeStruct((8, H), _f32),
        ],
    )(gA, gB, stA, stB, gam, bet, W1, b1, W2, b2)


def _layer_b_last_body(ga_ref, gb_ref, sta_ref, stb_ref, gam_ref, bet_ref,
                       w1_ref, b1_ref, w2_ref, b2_ref, gam2_ref, bet2_ref,
                       bi_ref, ow1_ref, ob1_ref, ow2_ref, ob2_ref, ow3_ref,
                       ob3_ref, z_ref, st2_s, pooled_s, cnt_s):
    hh = pl.program_id(0)
    ii = pl.program_id(1)
    u = _bn1_mlp(hh, ii, ga_ref, gb_ref, sta_ref, stb_ref, gam_ref, bet_ref,
                 w1_ref, b1_ref, w2_ref, b2_ref)

    @pl.when((hh == 0) & (ii == 0))
    def _():
        st2_s[...] = jnp.zeros((8, H), _f32)
        pooled_s[...] = jnp.zeros((G, H), _f32)
        cnt_s[...] = jnp.zeros((G, H), _f32)

    ones_row = jnp.full((1, Bn), 1.0, _f32)
    st2_s[0:1, :] += jnp.dot(ones_row, u, preferred_element_type=_f32)
    st2_s[1:2, :] += jnp.dot(ones_row, u * u, preferred_element_type=_f32)
    bi = bi_ref[...].reshape(1, Bn)
    oh = (lax.broadcasted_iota(jnp.int32, (G, Bn), 0) == bi).astype(_f32)
    pooled_s[...] += jnp.dot(oh, u, preferred_element_type=_f32)
    cnt_s[...] += jnp.sum(oh, axis=1, keepdims=True)

    @pl.when((hh == 1) & (ii == NBH - 1))
    def _():
        st = st2_s[...]
        mean = st[0:1, :] * (1.0 / N)
        var = st[1:2, :] * (1.0 / N) - mean * mean
        sc = lax.rsqrt(var + 1e-5) * gam2_ref[...]
        sh = bet2_ref[...] - mean * sc
        p = pooled_s[...] * sc + cnt_s[...][:, 0:1] * sh
        z1 = jnp.maximum(jnp.dot(p, ow1_ref[...], preferred_element_type=_f32)
                         + ob1_ref[...], 0.0)
        z2 = jnp.maximum(jnp.dot(z1, ow2_ref[...],
                                 preferred_element_type=_f32)
                         + ob2_ref[...], 0.0)
        z_ref[...] = ((z2 * ow3_ref[...]).sum(axis=1, keepdims=True)
                      + ob3_ref[...])


def _layer_b_last(gA, gB, stA, stB, gam, bet, W1, b1, W2, b2, gam2, bet2,
                  bidx3, oW1, ob1, oW2, ob2, oW3row, ob3s):
    return pl.pallas_call(
        _layer_b_last_body,
        grid=(2, NBH),
        in_specs=_B_IN_SPECS + [
            pl.BlockSpec((1, H), lambda h, i: (0, 0)),
            pl.BlockSpec((1, H), lambda h, i: (0, 0)),
            pl.BlockSpec((1, 1, Bn), lambda h, i: (h * NBH + i, 0, 0)),
            pl.BlockSpec((H, H // 2), lambda h, i: (0, 0)),
            pl.BlockSpec((1, H // 2), lambda h, i: (0, 0)),
            pl.BlockSpec((H // 2, H // 4), lambda h, i: (0, 0)),
            pl.BlockSpec((1, H // 4), lambda h, i: (0, 0)),
            pl.BlockSpec((1, H // 4), lambda h, i: (0, 0)),
            pl.BlockSpec((1, 1), lambda h, i: (0, 0)),
        ],
        out_specs=pl.BlockSpec((G, 1), lambda h, i: (0, 0)),
        out_shape=jax.ShapeDtypeStruct((G, 1), _f32),
        scratch_shapes=[
            pltpu.VMEM((8, H), _f32),
            pltpu.VMEM((G, H), _f32),
            pltpu.VMEM((G, H), _f32),
        ],
    )(gA, gB, stA, stB, gam, bet, W1, b1, W2, b2, gam2, bet2, bidx3,
      oW1, ob1, oW2, ob2, oW3row, ob3s)


def kernel(x, circle_index, parallel_node_index, batch_idx, edge_index,
           W_emb, b_emb, Wk, bk, Wself, bself, Wpar, bpar,
           bn1_g, bn1_b, mlpW1, mlpb1, mlpW2, mlpb2, bn2_g, bn2_b,
           oW1, ob1, oW2, ob2, oW3, ob3):
    del edge_index
    cid2 = circle_index.astype(jnp.int32).reshape(Nb, Bn * C)
    par2 = parallel_node_index.astype(jnp.int32).reshape(Nb, Bn)
    packed = jnp.concatenate(
        [cid2, par2, jnp.zeros((Nb, 8), jnp.int32)], axis=1)
    pad = jnp.zeros((HALF_PAD - HALF,), jnp.int32)
    idxA = jnp.concatenate([packed[:NBH].reshape(-1), pad])
    idxB = jnp.concatenate([packed[NBH:].reshape(-1), pad])
    bidx3 = batch_idx.astype(jnp.int32).reshape(Nb, 1, Bn)
    bk_sum = bk.sum(axis=1)
    bsp = bself + bpar
    ones1 = jnp.ones((1, H), _f32)
    zeros1 = jnp.zeros((1, H), _f32)
    stats_id = jnp.concatenate(
        [jnp.zeros((1, H), _f32),
         jnp.full((1, H), N * (1.0 - 1e-5), _f32),
         jnp.zeros((6, H), _f32)])

    u = _embed(x, W_emb, b_emb.reshape(1, H))
    st_prev, gam_prev, bet_prev = stats_id, ones1, zeros1
    gather = _sc_gather_fn()
    L = Wk.shape[0]
    for l in range(L):
        gthA = gather(u, idxA)
        gthB = gather(u, idxB)
        gA, stA = _layer_a_half(gthA, u, st_prev, gam_prev, bet_prev, Wk[l],
                                bk_sum[l].reshape(1, H), Wself[l], Wpar[l],
                                bsp[l].reshape(1, H), 0)
        gB, stB = _layer_a_half(gthB, u, st_prev, gam_prev, bet_prev, Wk[l],
                                bk_sum[l].reshape(1, H), Wself[l], Wpar[l],
                                bsp[l].reshape(1, H), 1)
        bargs = (gA, gB, stA, stB, bn1_g[l].reshape(1, H),
                 bn1_b[l].reshape(1, H), mlpW1[l], mlpb1[l].reshape(1, H),
                 mlpW2[l], mlpb2[l].reshape(1, H))
        if l == L - 1:
            return _layer_b_last(*bargs, bn2_g[l].reshape(1, H),
                                 bn2_b[l].reshape(1, H), bidx3, oW1,
                                 ob1.reshape(1, H // 2), oW2,
                                 ob2.reshape(1, H // 4),
                                 oW3.reshape(1, H // 4), ob3.reshape(1, 1))
        u, st2 = _layer_b(*bargs)
        st_prev, gam_prev, bet_prev = (st2, bn2_g[l].reshape(1, H),
                                       bn2_b[l].reshape(1, H))


# speedup vs baseline: 3.4112x; 1.0172x over previous
import functools

import jax
import jax.numpy as jnp
from jax import lax
from jax.experimental import pallas as pl
from jax.experimental.pallas import tpu as pltpu
from jax.experimental.pallas import tpu_sc as plsc

N = 10000
H = 128
C = 8
K = 3
G = 400
Bn = 1000
Nb = N // Bn
NBH = Nb // 2
GRP = Bn * C + Bn + 8
HALF = NBH * GRP
WORKERS = 32
PW = 1408
HALF_PAD = WORKERS * PW
RC = 352
NCHUNK = PW // RC

_f32 = jnp.float32
_bf16 = jnp.bfloat16


@functools.cache
def _sc_gather_fn():
    @functools.partial(
        pl.kernel,
        mesh=plsc.VectorSubcoreMesh(core_axis_name="c", subcore_axis_name="s"),
        out_type=jax.ShapeDtypeStruct((HALF_PAD, H), _f32),
    scratch_types=[
            pltpu.VMEM((PW,), jnp.int32),
            pltpu.VMEM((RC, H), _f32),
            pltpu.VMEM((RC, H), _f32),
            pltpu.SemaphoreType.DMA,
            pltpu.SemaphoreType.DMA,
            pltpu.SemaphoreType.DMA,
            pltpu.SemaphoreType.DMA,
        ],
    )
    def _sc_gather(u_hbm, idx_hbm, out_hbm, idx_v, rows0, rows1,
                   gs0, gs1, ws0, ws1):
        wid = lax.axis_index("s") * 2 + lax.axis_index("c")
        base = wid * PW
        bufs = [rows0, rows1]
        gsems = [gs0, gs1]
        wsems = [ws0, ws1]
        pltpu.sync_copy(idx_hbm.at[pl.ds(base, PW)], idx_v)
        gath = [None] * NCHUNK
        wr = [None] * NCHUNK
        gath[0] = pltpu.async_copy(
            u_hbm.at[idx_v.at[pl.ds(0, RC)]], bufs[0], gsems[0])
        for k in range(NCHUNK):
            gath[k].wait()
            if k + 1 < NCHUNK:
                if k >= 1:
                    wr[k - 1].wait()
                gath[k + 1] = pltpu.async_copy(
                    u_hbm.at[idx_v.at[pl.ds((k + 1) * RC, RC)]],
                    bufs[(k + 1) % 2], gsems[(k + 1) % 2])
            wr[k] = pltpu.async_copy(
                bufs[k % 2], out_hbm.at[pl.ds(base + k * RC, RC)],
                wsems[k % 2])
        wr[NCHUNK - 2].wait()
        wr[NCHUNK - 1].wait()

    return _sc_gather


def _embed_body(x_ref, w_ref, b_ref, o_ref):
    o_ref[...] = jnp.dot(x_ref[...], w_ref[...],
                         preferred_element_type=_f32) + b_ref[...]


def _embed(x, W, b2d):
    return pl.pallas_call(
        _embed_body,
        grid=(Nb,),
        in_specs=[
            pl.BlockSpec((Bn, H), lambda i: (i, 0)),
            pl.BlockSpec((H, H), lambda i: (0, 0)),
            pl.BlockSpec((1, H), lambda i: (0, 0)),
        ],
        out_specs=pl.BlockSpec((Bn, H), lambda i: (i, 0)),
        out_shape=jax.ShapeDtypeStruct((N, H), _f32),
    )(x, W, b2d)


def _elu(v):
    return jnp.where(v > 0, v, jnp.exp(v) - 1.0)


def _bn_scale(st_ref, gam_ref, bet_ref):
    mean = st_ref[0:1, :] * (1.0 / N)
    var = st_ref[1:2, :] * (1.0 / N) - mean * mean
    sc = lax.rsqrt(var + 1e-5) * gam_ref[...]
    return sc, bet_ref[...] - mean * sc


def _a_compute(gath_ref, u_ref, st_ref, gam_ref, bet_ref, wk_ref, bk_ref,
               ws_ref, wp_ref, bsp_ref):
    sc, sh = _bn_scale(st_ref, gam_ref, bet_ref)
    blk = gath_ref[...] * sc + sh
    neigh = blk[:Bn * C, :]
    hp = blk[Bn * C:Bn * C + Bn, :]
    h = u_ref[...] * sc + sh
    acc = jnp.zeros((Bn, C, H), _f32)
    for j in range(K):
        y = jnp.dot(neigh, wk_ref[j], preferred_element_type=_f32)
        y3 = y.reshape(Bn, C, H)
        if j:
            y3 = jnp.concatenate([y3[:, j:], y3[:, :j]], axis=1)
        acc = acc + y3
    acc = acc + bk_ref[...].reshape(1, 1, H)
    agg = _elu(acc).sum(axis=1)
    pre = (agg + jnp.dot(h, ws_ref[...], preferred_element_type=_f32)
           + jnp.dot(hp, wp_ref[...], preferred_element_type=_f32)
           + bsp_ref[...])
    return _elu(pre) + h


def _layer_a_body(gath_ref, u_ref, st_ref, gam_ref, bet_ref, wk_ref, bk_ref,
                  ws_ref, wp_ref, bsp_ref, g_ref, st1_ref):
    i = pl.program_id(0)
    g = _a_compute(gath_ref, u_ref, st_ref, gam_ref, bet_ref, wk_ref, bk_ref,
                   ws_ref, wp_ref, bsp_ref)
    g_ref[...] = g

    @pl.when(i == 0)
    def _():
        st1_ref[...] = jnp.zeros((8, H), _f32)

    ones_row = jnp.full((1, Bn), 1.0, _f32)
    st1_ref[0:1, :] += jnp.dot(ones_row, g, preferred_element_type=_f32)
    st1_ref[1:2, :] += jnp.dot(ones_row, g * g, preferred_element_type=_f32)


def _layer_a_half(gath, u, st_prev, gam_prev, bet_prev, Wk_l, bk_sum, Ws_l,
                  Wp_l, bsp, half):
    return pl.pallas_call(
        _layer_a_body,
        grid=(NBH,),
        in_specs=[
            pl.BlockSpec((GRP, H), lambda i: (i, 0)),
            pl.BlockSpec((Bn, H), lambda i: (i + half * NBH, 0)),
            pl.BlockSpec((8, H), lambda i: (0, 0)),
            pl.BlockSpec((1, H), lambda i: (0, 0)),
            pl.BlockSpec((1, H), lambda i: (0, 0)),
            pl.BlockSpec((K, H, H), lambda i: (0, 0, 0)),
            pl.BlockSpec((1, H), lambda i: (0, 0)),
            pl.BlockSpec((H, H), lambda i: (0, 0)),
            pl.BlockSpec((H, H), lambda i: (0, 0)),
            pl.BlockSpec((1, H), lambda i: (0, 0)),
        ],
        out_specs=[
            pl.BlockSpec((Bn, H), lambda i: (i, 0)),
            pl.BlockSpec((8, H), lambda i: (0, 0)),
        ],
        out_shape=[
            jax.ShapeDtypeStruct((NBH * Bn, H), _f32),
            jax.ShapeDtypeStruct((8, H), _f32),
        ],
    )(gath, u, st_prev, gam_prev, bet_prev, Wk_l, bk_sum, Ws_l, Wp_l, bsp)


NT = NBH + Nb


def _bn1_from(st, gam_ref, bet_ref):
    mean = st[0:1, :] * (1.0 / N)
    var = st[1:2, :] * (1.0 / N) - mean * mean
    sc = lax.rsqrt(var + 1e-5) * gam_ref[...]
    return sc, bet_ref[...] - mean * sc


def _ab_a_phase(s, gath_ref, u_ref, stp_ref, gamp_ref, betp_ref, wk_ref,
                bk_ref, ws_ref, wp_ref, bsp_ref, gb_s, stb_s):
    g = _a_compute(gath_ref, u_ref, stp_ref, gamp_ref, betp_ref, wk_ref,
                   bk_ref, ws_ref, wp_ref, bsp_ref)
    off = pl.multiple_of(s * Bn, Bn)
    gb_s[pl.ds(off, Bn), :] = g

    @pl.when(s == 0)
    def _():
        stb_s[...] = jnp.zeros((8, H), _f32)

    ones_row = jnp.full((1, Bn), 1.0, _f32)
    stb_s[0:1, :] += jnp.dot(ones_row, g, preferred_element_type=_f32)
    stb_s[1:2, :] += jnp.dot(ones_row, g * g, preferred_element_type=_f32)


def _ab_u(s, ga_ref, sta_ref, gam1_ref, bet1_ref, w1_ref, b1_ref, w2_ref,
          b2_ref, gb_s, stb_s):
    off = pl.multiple_of(jnp.maximum(s - 2 * NBH, 0) * Bn, Bn)
    gb = gb_s[pl.ds(off, Bn), :]
    g = jnp.where(s < 2 * NBH, ga_ref[...], gb)
    sc, sh = _bn1_from(sta_ref[...] + stb_s[...], gam1_ref, bet1_ref)
    gn = g * sc + sh
    m = jnp.maximum(jnp.dot(gn, w1_ref[...], preferred_element_type=_f32)
                    + b1_ref[...], 0.0)
    m = jnp.dot(m, w2_ref[...], preferred_element_type=_f32) + b2_ref[...]
    return gn + m


_AB_IN_SPECS = [
    pl.BlockSpec((GRP, H), lambda s: (jnp.minimum(s, NBH - 1), 0)),
    pl.BlockSpec((Bn, H), lambda s: (jnp.minimum(s + NBH, Nb - 1), 0)),
    pl.BlockSpec((8, H), lambda s: (0, 0)),
    pl.BlockSpec((1, H), lambda s: (0, 0)),
    pl.BlockSpec((1, H), lambda s: (0, 0)),
    pl.BlockSpec((K, H, H), lambda s: (0, 0, 0)),
    pl.BlockSpec((1, H), lambda s: (0, 0)),
    pl.BlockSpec((H, H), lambda s: (0, 0)),
    pl.BlockSpec((H, H), lambda s: (0, 0)),
    pl.BlockSpec((1, H), lambda s: (0, 0)),
    pl.BlockSpec((Bn, H), lambda s: (jnp.clip(s - NBH, 0, NBH - 1), 0)),
    pl.BlockSpec((8, H), lambda s: (0, 0)),
    pl.BlockSpec((1, H), lambda s: (0, 0)),
    pl.BlockSpec((1, H), lambda s: (0, 0)),
    pl.BlockSpec((H, H), lambda s: (0, 0)),
    pl.BlockSpec((1, H), lambda s: (0, 0)),
    pl.BlockSpec((H, H), lambda s: (0, 0)),
    pl.BlockSpec((1, H), lambda s: (0, 0)),
]


def _layer_ab_body(gath_ref, u_ref, stp_ref, gamp_ref, betp_ref, wk_ref,
                   bk_ref, ws_ref, wp_ref, bsp_ref, ga_ref, sta_ref,
                   gam1_ref, bet1_ref, w1_ref, b1_ref, w2_ref, b2_ref,
                   uo_ref, st2_ref, gb_s, stb_s):
    s = pl.program_id(0)

    @pl.when(s < NBH)
    def _():
        _ab_a_phase(s, gath_ref, u_ref, stp_ref, gamp_ref, betp_ref, wk_ref,
                    bk_ref, ws_ref, wp_ref, bsp_ref, gb_s, stb_s)

    @pl.when(s >= NBH)
    def _():
        u = _ab_u(s, ga_ref, sta_ref, gam1_ref, bet1_ref, w1_ref, b1_ref,
                  w2_ref, b2_ref, gb_s, stb_s)
        uo_ref[...] = u

        @pl.when(s == NBH)
        def _():
            st2_ref[...] = jnp.zeros((8, H), _f32)

        ones_row = jnp.full((1, Bn), 1.0, _f32)
        st2_ref[0:1, :] += jnp.dot(ones_row, u, preferred_element_type=_f32)
        st2_ref[1:2, :] += jnp.dot(ones_row, u * u,
                                   preferred_element_type=_f32)


def _layer_ab(gathB, u, stp, gamp, betp, Wk_l, bk_sum, Ws_l, Wp_l, bsp,
              gA, stA, gam1, bet1, W1, b1, W2, b2):
    return pl.pallas_call(
        _layer_ab_body,
        grid=(NT,),
        in_specs=_AB_IN_SPECS,
        out_specs=[
            pl.BlockSpec((Bn, H), lambda s: (jnp.clip(s - NBH, 0, Nb - 1), 0)),
            pl.BlockSpec((8, H), lambda s: (0, 0)),
        ],
        out_shape=[
            jax.ShapeDtypeStruct((N, H), _f32),
            jax.ShapeDtypeStruct((8, H), _f32),
        ],
        scratch_shapes=[
            pltpu.VMEM((NBH * Bn, H), _f32),
            pltpu.VMEM((8, H), _f32),
        ],
    )(gathB, u, stp, gamp, betp, Wk_l, bk_sum, Ws_l, Wp_l, bsp,
      gA, stA, gam1, bet1, W1, b1, W2, b2)


def _layer_ab_last_body(gath_ref, u_ref, stp_ref, gamp_ref, betp_ref, wk_ref,
                        bk_ref, ws_ref, wp_ref, bsp_ref, ga_ref, sta_ref,
                        gam1_ref, bet1_ref, w1_ref, b1_ref, w2_ref, b2_ref,
                        gam2_ref, bet2_ref, bi_ref, ow1_ref, ob1_ref, ow2_ref,
                        ob2_ref, ow3_ref, ob3_ref, z_ref, gb_s, stb_s, st2_s,
                        pooled_s, cnt_s):
    s = pl.program_id(0)

    @pl.when(s < NBH)
    def _():
        _ab_a_phase(s, gath_ref, u_ref, stp_ref, gamp_ref, betp_ref, wk_ref,
                    bk_ref, ws_ref, wp_ref, bsp_ref, gb_s, stb_s)

    @pl.when(s >= NBH)
    def _():
        u = _ab_u(s, ga_ref, sta_ref, gam1_ref, bet1_ref, w1_ref, b1_ref,
                  w2_ref, b2_ref, gb_s, stb_s)

        @pl.when(s == NBH)
        def _():
            st2_s[...] = jnp.zeros((8, H), _f32)
            pooled_s[...] = jnp.zeros((G, H), _f32)
            cnt_s[...] = jnp.zeros((G, H), _f32)

        ones_row = jnp.full((1, Bn), 1.0, _f32)
        st2_s[0:1, :] += jnp.dot(ones_row, u, preferred_element_type=_f32)
        st2_s[1:2, :] += jnp.dot(ones_row, u * u, preferred_element_type=_f32)
        bi = bi_ref[...].reshape(1, Bn)
        oh = (lax.broadcasted_iota(jnp.int32, (G, Bn), 0) == bi).astype(_f32)
        pooled_s[...] += jnp.dot(oh, u, preferred_element_type=_f32)
        cnt_s[...] += jnp.sum(oh, axis=1, keepdims=True)

        @pl.when(s == NT - 1)
        def _():
            st = st2_s[...]
            mean = st[0:1, :] * (1.0 / N)
            var = st[1:2, :] * (1.0 / N) - mean * mean
            sc = lax.rsqrt(var + 1e-5) * gam2_ref[...]
            sh = bet2_ref[...] - mean * sc
            p = pooled_s[...] * sc + cnt_s[...][:, 0:1] * sh
            z1 = jnp.maximum(
                jnp.dot(p, ow1_ref[...], preferred_element_type=_f32)
                + ob1_ref[...], 0.0)
            z2 = jnp.maximum(
                jnp.dot(z1, ow2_ref[...], preferred_element_type=_f32)
                + ob2_ref[...], 0.0)
            z_ref[...] = ((z2 * ow3_ref[...]).sum(axis=1, keepdims=True)
                          + ob3_ref[...])


def _layer_ab_last(gathB, u, stp, gamp, betp, Wk_l, bk_sum, Ws_l, Wp_l, bsp,
                   gA, stA, gam1, bet1, W1, b1, W2, b2, gam2, bet2, bidx3,
                   oW1, ob1, oW2, ob2, oW3row, ob3s):
    return pl.pallas_call(
        _layer_ab_last_body,
        grid=(NT,),
        in_specs=_AB_IN_SPECS + [
            pl.BlockSpec((1, H), lambda s: (0, 0)),
            pl.BlockSpec((1, H), lambda s: (0, 0)),
            pl.BlockSpec((1, 1, Bn), lambda s: (jnp.clip(s - NBH, 0, Nb - 1),
                                                0, 0)),
            pl.BlockSpec((H, H // 2), lambda s: (0, 0)),
            pl.BlockSpec((1, H // 2), lambda s: (0, 0)),
            pl.BlockSpec((H // 2, H // 4), lambda s: (0, 0)),
            pl.BlockSpec((1, H // 4), lambda s: (0, 0)),
            pl.BlockSpec((1, H // 4), lambda s: (0, 0)),
            pl.BlockSpec((1, 1), lambda s: (0, 0)),
        ],
        out_specs=pl.BlockSpec((G, 1), lambda s: (0, 0)),
        out_shape=jax.ShapeDtypeStruct((G, 1), _f32),
        scratch_shapes=[
            pltpu.VMEM((NBH * Bn, H), _f32),
            pltpu.VMEM((8, H), _f32),
            pltpu.VMEM((8, H), _f32),
            pltpu.VMEM((G, H), _f32),
            pltpu.VMEM((G, H), _f32),
        ],
    )(gathB, u, stp, gamp, betp, Wk_l, bk_sum, Ws_l, Wp_l, bsp,
      gA, stA, gam1, bet1, W1, b1, W2, b2, gam2, bet2, bidx3,
      oW1, ob1, oW2, ob2, oW3row, ob3s)


def kernel(x, circle_index, parallel_node_index, batch_idx, edge_index,
           W_emb, b_emb, Wk, bk, Wself, bself, Wpar, bpar,
           bn1_g, bn1_b, mlpW1, mlpb1, mlpW2, mlpb2, bn2_g, bn2_b,
           oW1, ob1, oW2, ob2, oW3, ob3):
    del edge_index
    cid2 = circle_index.astype(jnp.int32).reshape(Nb, Bn * C)
    par2 = parallel_node_index.astype(jnp.int32).reshape(Nb, Bn)
    packed = jnp.concatenate(
        [cid2, par2, jnp.zeros((Nb, 8), jnp.int32)], axis=1)
    pad = jnp.zeros((HALF_PAD - HALF,), jnp.int32)
    idxA = jnp.concatenate([packed[:NBH].reshape(-1), pad])
    idxB = jnp.concatenate([packed[NBH:].reshape(-1), pad])
    bidx3 = batch_idx.astype(jnp.int32).reshape(Nb, 1, Bn)
    bk_sum = bk.sum(axis=1)
    bsp = bself + bpar
    ones1 = jnp.ones((1, H), _f32)
    zeros1 = jnp.zeros((1, H), _f32)
    stats_id = jnp.concatenate(
        [jnp.zeros((1, H), _f32),
         jnp.full((1, H), N * (1.0 - 1e-5), _f32),
         jnp.zeros((6, H), _f32)])

    u = _embed(x, W_emb, b_emb.reshape(1, H))
    st_prev, gam_prev, bet_prev = stats_id, ones1, zeros1
    gather = _sc_gather_fn()
    L = Wk.shape[0]
    for l in range(L):
        gthA = gather(u, idxA)
        gthB = gather(u, idxB)
        gA, stA = _layer_a_half(gthA, u, st_prev, gam_prev, bet_prev, Wk[l],
                                bk_sum[l].reshape(1, H), Wself[l], Wpar[l],
                                bsp[l].reshape(1, H), 0)
        abargs = (gthB, u, st_prev, gam_prev, bet_prev, Wk[l],
                  bk_sum[l].reshape(1, H), Wself[l], Wpar[l],
                  bsp[l].reshape(1, H), gA, stA, bn1_g[l].reshape(1, H),
                  bn1_b[l].reshape(1, H), mlpW1[l], mlpb1[l].reshape(1, H),
                  mlpW2[l], mlpb2[l].reshape(1, H))
        if l == L - 1:
            return _layer_ab_last(*abargs, bn2_g[l].reshape(1, H),
                                  bn2_b[l].reshape(1, H), bidx3, oW1,
                                  ob1.reshape(1, H // 2), oW2,
                                  ob2.reshape(1, H // 4),
                                  oW3.reshape(1, H // 4), ob3.reshape(1, 1))
        u, st2 = _layer_ab(*abargs)
        st_prev, gam_prev, bet_prev = (st2, bn2_g[l].reshape(1, H),
                                       bn2_b[l].reshape(1, H))


# speedup vs baseline: 3.4760x; 1.0190x over previous
import functools

import jax
import jax.numpy as jnp
from jax import lax
from jax.experimental import pallas as pl
from jax.experimental.pallas import tpu as pltpu
from jax.experimental.pallas import tpu_sc as plsc

N = 10000
H = 128
C = 8
K = 3
G = 400
Bn = 1000
Nb = N // Bn
NBH = Nb // 2
GRP = Bn * C + Bn + 8
HALF = NBH * GRP
WORKERS = 32
PW = 1408
HALF_PAD = WORKERS * PW
RC = 352
NCHUNK = PW // RC

_f32 = jnp.float32
_bf16 = jnp.bfloat16


@functools.cache
def _sc_gather_fn():
    @functools.partial(
        pl.kernel,
        mesh=plsc.VectorSubcoreMesh(core_axis_name="c", subcore_axis_name="s"),
        out_type=jax.ShapeDtypeStruct((HALF_PAD, H), _f32),
    scratch_types=[
            pltpu.VMEM((PW,), jnp.int32),
            pltpu.VMEM((RC, H), _f32),
            pltpu.VMEM((RC, H), _f32),
            pltpu.SemaphoreType.DMA,
            pltpu.SemaphoreType.DMA,
            pltpu.SemaphoreType.DMA,
            pltpu.SemaphoreType.DMA,
        ],
    )
    def _sc_gather(u_hbm, idx_hbm, out_hbm, idx_v, rows0, rows1,
                   gs0, gs1, ws0, ws1):
        wid = lax.axis_index("s") * 2 + lax.axis_index("c")
        base = wid * PW
        bufs = [rows0, rows1]
        gsems = [gs0, gs1]
        wsems = [ws0, ws1]
        pltpu.sync_copy(idx_hbm.at[pl.ds(base, PW)], idx_v)
        gath = [None] * NCHUNK
        wr = [None] * NCHUNK
        gath[0] = pltpu.async_copy(
            u_hbm.at[idx_v.at[pl.ds(0, RC)]], bufs[0], gsems[0])
        for k in range(NCHUNK):
            gath[k].wait()
            if k + 1 < NCHUNK:
                if k >= 1:
                    wr[k - 1].wait()
                gath[k + 1] = pltpu.async_copy(
                    u_hbm.at[idx_v.at[pl.ds((k + 1) * RC, RC)]],
                    bufs[(k + 1) % 2], gsems[(k + 1) % 2])
            wr[k] = pltpu.async_copy(
                bufs[k % 2], out_hbm.at[pl.ds(base + k * RC, RC)],
                wsems[k % 2])
        wr[NCHUNK - 2].wait()
        wr[NCHUNK - 1].wait()

    return _sc_gather


def _embed_body(x_ref, w_ref, b_ref, o_ref):
    o_ref[...] = jnp.dot(x_ref[...], w_ref[...],
                         preferred_element_type=_f32) + b_ref[...]


def _embed(x, W, b2d):
    return pl.pallas_call(
        _embed_body,
        grid=(Nb,),
        in_specs=[
            pl.BlockSpec((Bn, H), lambda i: (i, 0)),
            pl.BlockSpec((H, H), lambda i: (0, 0)),
            pl.BlockSpec((1, H), lambda i: (0, 0)),
        ],
        out_specs=pl.BlockSpec((Bn, H), lambda i: (i, 0)),
        out_shape=jax.ShapeDtypeStruct((N, H), _f32),
    )(x, W, b2d)


def _elu(v):
    return jnp.where(v > 0, v, jnp.exp(v) - 1.0)


def _bn_scale(st_ref, gam_ref, bet_ref):
    mean = st_ref[0:1, :] * (1.0 / N)
    var = st_ref[1:2, :] * (1.0 / N) - mean * mean
    sc = lax.rsqrt(var + 1e-5) * gam_ref[...]
    return sc, bet_ref[...] - mean * sc


def _a_compute(gath_ref, u_ref, st_ref, gam_ref, bet_ref, wk_ref, bk_ref,
               ws_ref, wp_ref, bsp_ref):
    sc, sh = _bn_scale(st_ref, gam_ref, bet_ref)
    blk = gath_ref[...] * sc + sh
    neigh = blk[:Bn * C, :]
    hp = blk[Bn * C:Bn * C + Bn, :]
    h = u_ref[...] * sc + sh
    acc = jnp.zeros((Bn, C, H), _f32)
    for j in range(K):
        y = jnp.dot(neigh, wk_ref[j], preferred_element_type=_f32)
        y3 = y.reshape(Bn, C, H)
        if j:
            y3 = jnp.concatenate([y3[:, j:], y3[:, :j]], axis=1)
        acc = acc + y3
    acc = acc + bk_ref[...].reshape(1, 1, H)
    agg = _elu(acc).sum(axis=1)
    pre = (agg + jnp.dot(h, ws_ref[...], preferred_element_type=_f32)
           + jnp.dot(hp, wp_ref[...], preferred_element_type=_f32)
           + bsp_ref[...])
    return _elu(pre) + h


def _a_compute0(gath_ref, x_ref, we_ref, be_ref, wk_ref, bk_ref, ws_ref,
                wp_ref, bsp_ref):
    we = we_ref[...]
    be = be_ref[...]
    blk = gath_ref[...]
    neigh = blk[:Bn * C, :]
    hp = blk[Bn * C:Bn * C + Bn, :]
    h = jnp.dot(x_ref[...], we, preferred_element_type=_f32) + be
    wksum = wk_ref[0] + wk_ref[1] + wk_ref[2]
    kb = bk_ref[...] + jnp.dot(be, wksum, preferred_element_type=_f32)
    acc = jnp.zeros((Bn, C, H), _f32)
    for j in range(K):
        wkeff = jnp.dot(we, wk_ref[j], preferred_element_type=_f32)
        y = jnp.dot(neigh, wkeff, preferred_element_type=_f32)
        y3 = y.reshape(Bn, C, H)
        if j:
            y3 = jnp.concatenate([y3[:, j:], y3[:, :j]], axis=1)
        acc = acc + y3
    acc = acc + kb.reshape(1, 1, H)
    agg = _elu(acc).sum(axis=1)
    wp = wp_ref[...]
    wpeff = jnp.dot(we, wp, preferred_element_type=_f32)
    pre = (agg + jnp.dot(h, ws_ref[...], preferred_element_type=_f32)
           + jnp.dot(hp, wpeff, preferred_element_type=_f32)
           + bsp_ref[...] + jnp.dot(be, wp, preferred_element_type=_f32))
    return _elu(pre) + h


def _layer_a_body(gath_ref, u_ref, st_ref, gam_ref, bet_ref, wk_ref, bk_ref,
                  ws_ref, wp_ref, bsp_ref, g_ref, st1_ref):
    i = pl.program_id(0)
    g = _a_compute(gath_ref, u_ref, st_ref, gam_ref, bet_ref, wk_ref, bk_ref,
                   ws_ref, wp_ref, bsp_ref)
    g_ref[...] = g

    @pl.when(i == 0)
    def _():
        st1_ref[...] = jnp.zeros((8, H), _f32)

    ones_row = jnp.full((1, Bn), 1.0, _f32)
    st1_ref[0:1, :] += jnp.dot(ones_row, g, preferred_element_type=_f32)
    st1_ref[1:2, :] += jnp.dot(ones_row, g * g, preferred_element_type=_f32)


def _layer_a_half(gath, u, st_prev, gam_prev, bet_prev, Wk_l, bk_sum, Ws_l,
                  Wp_l, bsp, half):
    return pl.pallas_call(
        _layer_a_body,
        grid=(NBH,),
        in_specs=[
            pl.BlockSpec((GRP, H), lambda i: (i, 0)),
            pl.BlockSpec((Bn, H), lambda i: (i + half * NBH, 0)),
            pl.BlockSpec((8, H), lambda i: (0, 0)),
            pl.BlockSpec((1, H), lambda i: (0, 0)),
            pl.BlockSpec((1, H), lambda i: (0, 0)),
            pl.BlockSpec((K, H, H), lambda i: (0, 0, 0)),
            pl.BlockSpec((1, H), lambda i: (0, 0)),
            pl.BlockSpec((H, H), lambda i: (0, 0)),
            pl.BlockSpec((H, H), lambda i: (0, 0)),
            pl.BlockSpec((1, H), lambda i: (0, 0)),
        ],
        out_specs=[
            pl.BlockSpec((Bn, H), lambda i: (i, 0)),
            pl.BlockSpec((8, H), lambda i: (0, 0)),
        ],
        out_shape=[
            jax.ShapeDtypeStruct((NBH * Bn, H), _f32),
            jax.ShapeDtypeStruct((8, H), _f32),
        ],
    )(gath, u, st_prev, gam_prev, bet_prev, Wk_l, bk_sum, Ws_l, Wp_l, bsp)


NT = NBH + Nb


def _layer_a0_body(gath_ref, x_ref, we_ref, be_ref, wk_ref, bk_ref,
                   ws_ref, wp_ref, bsp_ref, g_ref, st1_ref):
    i = pl.program_id(0)
    g = _a_compute0(gath_ref, x_ref, we_ref, be_ref, wk_ref, bk_ref,
                    ws_ref, wp_ref, bsp_ref)
    g_ref[...] = g

    @pl.when(i == 0)
    def _():
        st1_ref[...] = jnp.zeros((8, H), _f32)

    ones_row = jnp.full((1, Bn), 1.0, _f32)
    st1_ref[0:1, :] += jnp.dot(ones_row, g, preferred_element_type=_f32)
    st1_ref[1:2, :] += jnp.dot(ones_row, g * g, preferred_element_type=_f32)


def _layer_a_half0(gath, x, We, be2, Wk_l, bk_sum, Ws_l, Wp_l, bsp, half):
    return pl.pallas_call(
        _layer_a0_body,
        grid=(NBH,),
        in_specs=[
            pl.BlockSpec((GRP, H), lambda i: (i, 0)),
            pl.BlockSpec((Bn, H), lambda i: (i + half * NBH, 0)),
            pl.BlockSpec((H, H), lambda i: (0, 0)),
            pl.BlockSpec((1, H), lambda i: (0, 0)),
            pl.BlockSpec((K, H, H), lambda i: (0, 0, 0)),
            pl.BlockSpec((1, H), lambda i: (0, 0)),
            pl.BlockSpec((H, H), lambda i: (0, 0)),
            pl.BlockSpec((H, H), lambda i: (0, 0)),
            pl.BlockSpec((1, H), lambda i: (0, 0)),
        ],
        out_specs=[
            pl.BlockSpec((Bn, H), lambda i: (i, 0)),
            pl.BlockSpec((8, H), lambda i: (0, 0)),
        ],
        out_shape=[
            jax.ShapeDtypeStruct((NBH * Bn, H), _f32),
            jax.ShapeDtypeStruct((8, H), _f32),
        ],
    )(gath, x, We, be2, Wk_l, bk_sum, Ws_l, Wp_l, bsp)


def _bn1_from(st, gam_ref, bet_ref):
    mean = st[0:1, :] * (1.0 / N)
    var = st[1:2, :] * (1.0 / N) - mean * mean
    sc = lax.rsqrt(var + 1e-5) * gam_ref[...]
    return sc, bet_ref[...] - mean * sc


def _ab_a_phase(s, gath_ref, u_ref, stp_ref, gamp_ref, betp_ref, wk_ref,
                bk_ref, ws_ref, wp_ref, bsp_ref, gb_s, stb_s):
    g = _a_compute(gath_ref, u_ref, stp_ref, gamp_ref, betp_ref, wk_ref,
                   bk_ref, ws_ref, wp_ref, bsp_ref)
    off = pl.multiple_of(s * Bn, Bn)
    gb_s[pl.ds(off, Bn), :] = g

    @pl.when(s == 0)
    def _():
        stb_s[...] = jnp.zeros((8, H), _f32)

    ones_row = jnp.full((1, Bn), 1.0, _f32)
    stb_s[0:1, :] += jnp.dot(ones_row, g, preferred_element_type=_f32)
    stb_s[1:2, :] += jnp.dot(ones_row, g * g, preferred_element_type=_f32)


def _ab_u(s, ga_ref, sta_ref, gam1_ref, bet1_ref, w1_ref, b1_ref, w2_ref,
          b2_ref, gb_s, stb_s):
    off = pl.multiple_of(jnp.maximum(s - 2 * NBH, 0) * Bn, Bn)
    gb = gb_s[pl.ds(off, Bn), :]
    g = jnp.where(s < 2 * NBH, ga_ref[...], gb)
    sc, sh = _bn1_from(sta_ref[...] + stb_s[...], gam1_ref, bet1_ref)
    gn = g * sc + sh
    m = jnp.maximum(jnp.dot(gn, w1_ref[...], preferred_element_type=_f32)
                    + b1_ref[...], 0.0)
    m = jnp.dot(m, w2_ref[...], preferred_element_type=_f32) + b2_ref[...]
    return gn + m


_AB_IN_SPECS = [
    pl.BlockSpec((GRP, H), lambda s: (jnp.minimum(s, NBH - 1), 0)),
    pl.BlockSpec((Bn, H), lambda s: (jnp.minimum(s + NBH, Nb - 1), 0)),
    pl.BlockSpec((8, H), lambda s: (0, 0)),
    pl.BlockSpec((1, H), lambda s: (0, 0)),
    pl.BlockSpec((1, H), lambda s: (0, 0)),
    pl.BlockSpec((K, H, H), lambda s: (0, 0, 0)),
    pl.BlockSpec((1, H), lambda s: (0, 0)),
    pl.BlockSpec((H, H), lambda s: (0, 0)),
    pl.BlockSpec((H, H), lambda s: (0, 0)),
    pl.BlockSpec((1, H), lambda s: (0, 0)),
    pl.BlockSpec((Bn, H), lambda s: (jnp.clip(s - NBH, 0, NBH - 1), 0)),
    pl.BlockSpec((8, H), lambda s: (0, 0)),
    pl.BlockSpec((1, H), lambda s: (0, 0)),
    pl.BlockSpec((1, H), lambda s: (0, 0)),
    pl.BlockSpec((H, H), lambda s: (0, 0)),
    pl.BlockSpec((1, H), lambda s: (0, 0)),
    pl.BlockSpec((H, H), lambda s: (0, 0)),
    pl.BlockSpec((1, H), lambda s: (0, 0)),
]


def _layer_ab_body(gath_ref, u_ref, stp_ref, gamp_ref, betp_ref, wk_ref,
                   bk_ref, ws_ref, wp_ref, bsp_ref, ga_ref, sta_ref,
                   gam1_ref, bet1_ref, w1_ref, b1_ref, w2_ref, b2_ref,
                   uo_ref, st2_ref, gb_s, stb_s):
    s = pl.program_id(0)

    @pl.when(s < NBH)
    def _():
        _ab_a_phase(s, gath_ref, u_ref, stp_ref, gamp_ref, betp_ref, wk_ref,
                    bk_ref, ws_ref, wp_ref, bsp_ref, gb_s, stb_s)

    @pl.when(s >= NBH)
    def _():
        u = _ab_u(s, ga_ref, sta_ref, gam1_ref, bet1_ref, w1_ref, b1_ref,
                  w2_ref, b2_ref, gb_s, stb_s)
        uo_ref[...] = u

        @pl.when(s == NBH)
        def _():
            st2_ref[...] = jnp.zeros((8, H), _f32)

        ones_row = jnp.full((1, Bn), 1.0, _f32)
        st2_ref[0:1, :] += jnp.dot(ones_row, u, preferred_element_type=_f32)
        st2_ref[1:2, :] += jnp.dot(ones_row, u * u,
                                   preferred_element_type=_f32)


def _layer_ab(gathB, u, stp, gamp, betp, Wk_l, bk_sum, Ws_l, Wp_l, bsp,
              gA, stA, gam1, bet1, W1, b1, W2, b2):
    return pl.pallas_call(
        _layer_ab_body,
        grid=(NT,),
        in_specs=_AB_IN_SPECS,
        out_specs=[
            pl.BlockSpec((Bn, H), lambda s: (jnp.clip(s - NBH, 0, Nb - 1), 0)),
            pl.BlockSpec((8, H), lambda s: (0, 0)),
        ],
        out_shape=[
            jax.ShapeDtypeStruct((N, H), _f32),
            jax.ShapeDtypeStruct((8, H), _f32),
        ],
        scratch_shapes=[
            pltpu.VMEM((NBH * Bn, H), _f32),
            pltpu.VMEM((8, H), _f32),
        ],
    )(gathB, u, stp, gamp, betp, Wk_l, bk_sum, Ws_l, Wp_l, bsp,
      gA, stA, gam1, bet1, W1, b1, W2, b2)


def _layer_ab0_body(gath_ref, x_ref, we_ref, be_ref, wk_ref,
                    bk_ref, ws_ref, wp_ref, bsp_ref, ga_ref, sta_ref,
                    gam1_ref, bet1_ref, w1_ref, b1_ref, w2_ref, b2_ref,
                    uo_ref, st2_ref, gb_s, stb_s):
    s = pl.program_id(0)

    @pl.when(s < NBH)
    def _():
        g = _a_compute0(gath_ref, x_ref, we_ref, be_ref, wk_ref, bk_ref,
                        ws_ref, wp_ref, bsp_ref)
        off = pl.multiple_of(s * Bn, Bn)
        gb_s[pl.ds(off, Bn), :] = g

        @pl.when(s == 0)
        def _():
            stb_s[...] = jnp.zeros((8, H), _f32)

        ones_row = jnp.full((1, Bn), 1.0, _f32)
        stb_s[0:1, :] += jnp.dot(ones_row, g, preferred_element_type=_f32)
        stb_s[1:2, :] += jnp.dot(ones_row, g * g, preferred_element_type=_f32)

    @pl.when(s >= NBH)
    def _():
        u = _ab_u(s, ga_ref, sta_ref, gam1_ref, bet1_ref, w1_ref, b1_ref,
                  w2_ref, b2_ref, gb_s, stb_s)
        uo_ref[...] = u

        @pl.when(s == NBH)
        def _():
            st2_ref[...] = jnp.zeros((8, H), _f32)

        ones_row = jnp.full((1, Bn), 1.0, _f32)
        st2_ref[0:1, :] += jnp.dot(ones_row, u, preferred_element_type=_f32)
        st2_ref[1:2, :] += jnp.dot(ones_row, u * u,
                                   preferred_element_type=_f32)


def _layer_ab0(gathB, x, We, be2, Wk_l, bk_sum, Ws_l, Wp_l, bsp,
               gA, stA, gam1, bet1, W1, b1, W2, b2):
    in_specs = [
        pl.BlockSpec((GRP, H), lambda s: (jnp.minimum(s, NBH - 1), 0)),
        pl.BlockSpec((Bn, H), lambda s: (jnp.minimum(s + NBH, Nb - 1), 0)),
        pl.BlockSpec((H, H), lambda s: (0, 0)),
        pl.BlockSpec((1, H), lambda s: (0, 0)),
        pl.BlockSpec((K, H, H), lambda s: (0, 0, 0)),
        pl.BlockSpec((1, H), lambda s: (0, 0)),
        pl.BlockSpec((H, H), lambda s: (0, 0)),
        pl.BlockSpec((H, H), lambda s: (0, 0)),
        pl.BlockSpec((1, H), lambda s: (0, 0)),
        pl.BlockSpec((Bn, H), lambda s: (jnp.clip(s - NBH, 0, NBH - 1), 0)),
        pl.BlockSpec((8, H), lambda s: (0, 0)),
        pl.BlockSpec((1, H), lambda s: (0, 0)),
        pl.BlockSpec((1, H), lambda s: (0, 0)),
        pl.BlockSpec((H, H), lambda s: (0, 0)),
        pl.BlockSpec((1, H), lambda s: (0, 0)),
        pl.BlockSpec((H, H), lambda s: (0, 0)),
        pl.BlockSpec((1, H), lambda s: (0, 0)),
    ]
    return pl.pallas_call(
        _layer_ab0_body,
        grid=(NT,),
        in_specs=in_specs,
        out_specs=[
            pl.BlockSpec((Bn, H), lambda s: (jnp.clip(s - NBH, 0, Nb - 1), 0)),
            pl.BlockSpec((8, H), lambda s: (0, 0)),
        ],
        out_shape=[
            jax.ShapeDtypeStruct((N, H), _f32),
            jax.ShapeDtypeStruct((8, H), _f32),
        ],
        scratch_shapes=[
            pltpu.VMEM((NBH * Bn, H), _f32),
            pltpu.VMEM((8, H), _f32),
        ],
    )(gathB, x, We, be2, Wk_l, bk_sum, Ws_l, Wp_l, bsp,
      gA, stA, gam1, bet1, W1, b1, W2, b2)


def _layer_ab_last_body(gath_ref, u_ref, stp_ref, gamp_ref, betp_ref, wk_ref,
                        bk_ref, ws_ref, wp_ref, bsp_ref, ga_ref, sta_ref,
                        gam1_ref, bet1_ref, w1_ref, b1_ref, w2_ref, b2_ref,
                        gam2_ref, bet2_ref, bi_ref, ow1_ref, ob1_ref, ow2_ref,
                        ob2_ref, ow3_ref, ob3_ref, z_ref, gb_s, stb_s, st2_s,
                        pooled_s, cnt_s):
    s = pl.program_id(0)

    @pl.when(s < NBH)
    def _():
        _ab_a_phase(s, gath_ref, u_ref, stp_ref, gamp_ref, betp_ref, wk_ref,
                    bk_ref, ws_ref, wp_ref, bsp_ref, gb_s, stb_s)

    @pl.when(s >= NBH)
    def _():
        u = _ab_u(s, ga_ref, sta_ref, gam1_ref, bet1_ref, w1_ref, b1_ref,
                  w2_ref, b2_ref, gb_s, stb_s)

        @pl.when(s == NBH)
        def _():
            st2_s[...] = jnp.zeros((8, H), _f32)
            pooled_s[...] = jnp.zeros((G, H), _f32)
            cnt_s[...] = jnp.zeros((G, H), _f32)

        ones_row = jnp.full((1, Bn), 1.0, _f32)
        st2_s[0:1, :] += jnp.dot(ones_row, u, preferred_element_type=_f32)
        st2_s[1:2, :] += jnp.dot(ones_row, u * u, preferred_element_type=_f32)
        bi = bi_ref[...].reshape(1, Bn)
        oh = (lax.broadcasted_iota(jnp.int32, (G, Bn), 0) == bi).astype(_f32)
        pooled_s[...] += jnp.dot(oh, u, preferred_element_type=_f32)
        cnt_s[...] += jnp.sum(oh, axis=1, keepdims=True)

        @pl.when(s == NT - 1)
        def _():
            st = st2_s[...]
            mean = st[0:1, :] * (1.0 / N)
            var = st[1:2, :] * (1.0 / N) - mean * mean
            sc = lax.rsqrt(var + 1e-5) * gam2_ref[...]
            sh = bet2_ref[...] - mean * sc
            p = pooled_s[...] * sc + cnt_s[...][:, 0:1] * sh
            z1 = jnp.maximum(
                jnp.dot(p, ow1_ref[...], preferred_element_type=_f32)
                + ob1_ref[...], 0.0)
            z2 = jnp.maximum(
                jnp.dot(z1, ow2_ref[...], preferred_element_type=_f32)
                + ob2_ref[...], 0.0)
            z_ref[...] = ((z2 * ow3_ref[...]).sum(axis=1, keepdims=True)
                          + ob3_ref[...])


def _layer_ab_last(gathB, u, stp, gamp, betp, Wk_l, bk_sum, Ws_l, Wp_l, bsp,
                   gA, stA, gam1, bet1, W1, b1, W2, b2, gam2, bet2, bidx3,
                   oW1, ob1, oW2, ob2, oW3row, ob3s):
    return pl.pallas_call(
        _layer_ab_last_body,
        grid=(NT,),
        in_specs=_AB_IN_SPECS + [
            pl.BlockSpec((1, H), lambda s: (0, 0)),
            pl.BlockSpec((1, H), lambda s: (0, 0)),
            pl.BlockSpec((1, 1, Bn), lambda s: (jnp.clip(s - NBH, 0, Nb - 1),
                                                0, 0)),
            pl.BlockSpec((H, H // 2), lambda s: (0, 0)),
            pl.BlockSpec((1, H // 2), lambda s: (0, 0)),
            pl.BlockSpec((H // 2, H // 4), lambda s: (0, 0)),
            pl.BlockSpec((1, H // 4), lambda s: (0, 0)),
            pl.BlockSpec((1, H // 4), lambda s: (0, 0)),
            pl.BlockSpec((1, 1), lambda s: (0, 0)),
        ],
        out_specs=pl.BlockSpec((G, 1), lambda s: (0, 0)),
        out_shape=jax.ShapeDtypeStruct((G, 1), _f32),
        scratch_shapes=[
            pltpu.VMEM((NBH * Bn, H), _f32),
            pltpu.VMEM((8, H), _f32),
            pltpu.VMEM((8, H), _f32),
            pltpu.VMEM((G, H), _f32),
            pltpu.VMEM((G, H), _f32),
        ],
    )(gathB, u, stp, gamp, betp, Wk_l, bk_sum, Ws_l, Wp_l, bsp,
      gA, stA, gam1, bet1, W1, b1, W2, b2, gam2, bet2, bidx3,
      oW1, ob1, oW2, ob2, oW3row, ob3s)


def kernel(x, circle_index, parallel_node_index, batch_idx, edge_index,
           W_emb, b_emb, Wk, bk, Wself, bself, Wpar, bpar,
           bn1_g, bn1_b, mlpW1, mlpb1, mlpW2, mlpb2, bn2_g, bn2_b,
           oW1, ob1, oW2, ob2, oW3, ob3):
    del edge_index
    cid2 = circle_index.astype(jnp.int32).reshape(Nb, Bn * C)
    par2 = parallel_node_index.astype(jnp.int32).reshape(Nb, Bn)
    packed = jnp.concatenate(
        [cid2, par2, jnp.zeros((Nb, 8), jnp.int32)], axis=1)
    pad = jnp.zeros((HALF_PAD - HALF,), jnp.int32)
    idxA = jnp.concatenate([packed[:NBH].reshape(-1), pad])
    idxB = jnp.concatenate([packed[NBH:].reshape(-1), pad])
    bidx3 = batch_idx.astype(jnp.int32).reshape(Nb, 1, Bn)
    bk_sum = bk.sum(axis=1)
    bsp = bself + bpar
    ones1 = jnp.ones((1, H), _f32)
    zeros1 = jnp.zeros((1, H), _f32)
    stats_id = jnp.concatenate(
        [jnp.zeros((1, H), _f32),
         jnp.full((1, H), N * (1.0 - 1e-5), _f32),
         jnp.zeros((6, H), _f32)])

    st_prev, gam_prev, bet_prev = stats_id, ones1, zeros1
    gather = _sc_gather_fn()
    L = Wk.shape[0]
    u = x
    for l in range(L):
        gthA = gather(u, idxA)
        gthB = gather(u, idxB)
        if l == 0:
            a0args = (u, W_emb, b_emb.reshape(1, H), Wk[l],
                      bk_sum[l].reshape(1, H), Wself[l], Wpar[l],
                      bsp[l].reshape(1, H))
            gA, stA = _layer_a_half0(gthA, *a0args, 0)
            u, st2 = _layer_ab0(gthB, *a0args, gA, stA,
                                bn1_g[l].reshape(1, H),
                                bn1_b[l].reshape(1, H), mlpW1[l],
                                mlpb1[l].reshape(1, H), mlpW2[l],
                                mlpb2[l].reshape(1, H))
            st_prev, gam_prev, bet_prev = (st2, bn2_g[l].reshape(1, H),
                                           bn2_b[l].reshape(1, H))
            continue
        gA, stA = _layer_a_half(gthA, u, st_prev, gam_prev, bet_prev, Wk[l],
                                bk_sum[l].reshape(1, H), Wself[l], Wpar[l],
                                bsp[l].reshape(1, H), 0)
        abargs = (gthB, u, st_prev, gam_prev, bet_prev, Wk[l],
                  bk_sum[l].reshape(1, H), Wself[l], Wpar[l],
                  bsp[l].reshape(1, H), gA, stA, bn1_g[l].reshape(1, H),
                  bn1_b[l].reshape(1, H), mlpW1[l], mlpb1[l].reshape(1, H),
                  mlpW2[l], mlpb2[l].reshape(1, H))
        if l == L - 1:
            return _layer_ab_last(*abargs, bn2_g[l].reshape(1, H),
                                  bn2_b[l].reshape(1, H), bidx3, oW1,
                                  ob1.reshape(1, H // 2), oW2,
                                  ob2.reshape(1, H // 4),
                                  oW3.reshape(1, H // 4), ob3.reshape(1, 1))
        u, st2 = _layer_ab(*abargs)
        st_prev, gam_prev, bet_prev = (st2, bn2_g[l].reshape(1, H),
                                       bn2_b[l].reshape(1, H))
